# baseline (speedup 1.0000x reference)
import functools

import jax
import jax.numpy as jnp
from jax import lax
from jax.experimental import pallas as pl
from jax.experimental.pallas import tpu as pltpu
from jax.experimental.pallas import tpu_sc as plsc

N, E, B = 10000, 160000, 512
NP = 10240
SN = 2 * NP
EP = 163840
SE = 2 * EP
NC, NS = 2, 16
EPT = EP // NS
KC = 64
NPT = NP // NS
NB = 640
EPS = 1e-9
RB = 2048
REB = 2048

@functools.cache
def _get_mesh():
    return plsc.VectorSubcoreMesh(core_axis_name="c", subcore_axis_name="s",
                                  num_cores=NC, num_subcores=NS)


def _leaky(x):
    return jnp.where(x >= 0, x, 0.01 * x)


def _elu(x):
    return jnp.where(x > 0, x, jnp.exp(x) - 1.0)


def _gru_tc(x, h, wihT, whhT, bih, bhh):
    gi = jnp.dot(x, wihT, preferred_element_type=jnp.float32) + bih
    gh = jnp.dot(h, whhT, preferred_element_type=jnp.float32) + bhh
    r = jax.nn.sigmoid(gi[:, 0:128] + gh[:, 0:128])
    z = jax.nn.sigmoid(gi[:, 128:256] + gh[:, 128:256])
    n = jnp.tanh(gi[:, 256:384] + r * gh[:, 256:384])
    return (1.0 - z) * n + z * h


def _sck_gather_rows(table, idx, d, dtype=jnp.float32):
    kc = 256
    PW0 = 14336
    PW1 = (SE - 16 * PW0) // 16
    EC0 = 16 * PW0

    def body(tab_h, idx_h, out_h, idx0, idx1, rows0, rows1,
             g0, g1, s0, s1):
        c = lax.axis_index("c")
        t = lax.axis_index("s")
        base = jnp.where(c == 0, t * PW0, EC0 + t * PW1)
        nch = jnp.where(c == 0, PW0 // kc, PW1 // kc)
        idx_b = [idx0, idx1]
        rows_b = [rows0, rows1]
        gsem = [g0, g1]
        ssem = [s0, s1]

        def start_gather(k, b):
            pltpu.sync_copy(idx_h.at[pl.ds(base + k * kc, kc)], idx_b[b])
            pltpu.async_copy(tab_h.at[idx_b[b]], rows_b[b], gsem[b])

        start_gather(0, 0)

        def pair(g, _):
            for b in range(2):
                k = g * 2 + b
                nb = 1 - b
                pltpu.make_async_copy(tab_h.at[idx_b[b]], rows_b[b],
                                      gsem[b]).wait()

                @pl.when(k + 1 < nch)
                def _pre():
                    @pl.when(k >= 1)
                    def _drain():
                        pltpu.make_async_copy(
                            rows_b[nb], out_h.at[pl.ds(base, kc)],
                            ssem[nb]).wait()
                    start_gather(k + 1, nb)

                pltpu.async_copy(rows_b[b], out_h.at[pl.ds(base + k * kc, kc)],
                                 ssem[b])
            return 0

        lax.fori_loop(0, nch // 2, pair, 0)
        pltpu.make_async_copy(rows_b[0], out_h.at[pl.ds(base, kc)],
                              ssem[0]).wait()
        pltpu.make_async_copy(rows_b[1], out_h.at[pl.ds(base, kc)],
                              ssem[1]).wait()

    f = pl.kernel(
        body,
        out_type=jax.ShapeDtypeStruct((SE, d), dtype),
        mesh=_get_mesh(),
        compiler_params=pltpu.CompilerParams(needs_layout_passes=False),
        scratch_types=[
            pltpu.VMEM((kc,), jnp.int32),
            pltpu.VMEM((kc,), jnp.int32),
            pltpu.VMEM((kc, d), dtype),
            pltpu.VMEM((kc, d), dtype),
            pltpu.SemaphoreType.DMA,
            pltpu.SemaphoreType.DMA,
            pltpu.SemaphoreType.DMA,
            pltpu.SemaphoreType.DMA,
        ],
    )
    return f(table, idx)


def _scale_loop(rows_v, wv, kc, dcols):
    def sbody(j, _):
        wj = plsc.load_gather(wv, [jnp.full((16,), j, jnp.int32)])
        for f in range(dcols // 16):
            sl = pl.ds(f * 16, 16)
            rows_v[j, sl] = rows_v[j, sl] * wj
        return 0

    lax.fori_loop(0, kc, sbody, 0)


GN = 4
PV = 10112


def _sck_edge(gather_rows):
    kc = 32 if gather_rows else KC
    nch = EPT // kc
    ngrp = nch // GN
    CPT = EPT // kc

    def body(rows_h, q_h, p_h, dst_h, src_h, z128_h, z1_h, outC, outS, outSP,
             p_v, ps_v, idxd0, idxd1, idxs0, idxs1, rows0, rows1, qg0, qg1,
             wv, s_v, acc_sh, i0, i1, g0, g1, c0, c1):
        c = lax.axis_index("c")
        t = lax.axis_index("s")
        idxd_g = [idxd0, idxd1]
        idxs_g = [idxs0, idxs1]
        rows_b = [rows0, rows1]
        qg_b = [qg0, qg1]
        isem = [i0, i1]
        gsem = [g0, g1]
        csem = [c0, c1]
        pltpu.sync_copy(z128_h.at[pl.ds(t * NPT, NPT)],
                        acc_sh.at[pl.ds(t * NPT, NPT)])
        pltpu.sync_copy(z1_h, s_v)
        pltpu.sync_copy(p_h.at[c].at[pl.ds(0, PV)], p_v)
        if gather_rows:
            pltpu.sync_copy(q_h.at[c].at[pl.ds(0, PV)], ps_v)
        plsc.subcore_barrier()

        def g_idx_load(m, buf):
            crow = t * CPT + m * GN
            frow = c * CPT * NS + crow
            pltpu.async_copy(dst_h.at[c].at[pl.ds(crow, GN)], idxd_g[buf],
                             isem[buf])
            if gather_rows:
                pltpu.async_copy(src_h.at[pl.ds(frow, GN)], idxs_g[buf],
                                 isem[buf])
            else:
                pltpu.async_copy(q_h.at[pl.ds(frow, GN)], qg_b[buf],
                                 isem[buf])

        def wait_gidx(buf):
            pltpu.make_async_copy(dst_h.at[c].at[pl.ds(0, GN)], idxd_g[buf],
                                  isem[buf]).wait()
            if gather_rows:
                pltpu.make_async_copy(src_h.at[pl.ds(0, GN)], idxs_g[buf],
                                      isem[buf]).wait()
            else:
                pltpu.make_async_copy(q_h.at[pl.ds(0, GN)], qg_b[buf],
                                      isem[buf]).wait()

        def start_rows(k, gb, j, rb):
            if gather_rows:
                pltpu.async_copy(rows_h.at[idxs_g[gb].at[j]], rows_b[rb],
                                 gsem[rb])
            else:
                fbase = c * EP + t * EPT + k * kc
                pltpu.async_copy(rows_h.at[pl.ds(fbase, KC)], rows_b[rb],
                                 gsem[rb])

        def wait_rows(rb):
            if gather_rows:
                pltpu.make_async_copy(rows_h.at[idxs_g[0].at[0]], rows_b[rb],
                                      gsem[rb]).wait()
            else:
                pltpu.make_async_copy(rows_h.at[pl.ds(0, kc)], rows_b[rb],
                                      gsem[rb]).wait()

        def wait_scat(rb):
            pltpu.make_async_copy(rows_b[rb], acc_sh.at[idxd_g[0].at[0]],
                                  csem[rb]).wait()

        g_idx_load(0, 0)
        wait_gidx(0)
        start_rows(0, 0, 0, 0)
        g_idx_load(1, 1)

        def gpair(gp, _):
            for gpar in range(2):
                g = gp * 2 + gpar
                gb = gpar
                for j in range(GN):
                    k = g * GN + j
                    rb = j % 2
                    nrb = 1 - rb
                    wait_rows(rb)
                    if j == 1:
                        @pl.when(jnp.logical_and(g >= 1, g + 1 < ngrp))
                        def _ld():
                            g_idx_load(g + 1, 1 - gb)

                    @pl.when(k + 1 < nch)
                    def _pre():
                        @pl.when(k >= 1)
                        def _drain():
                            wait_scat(nrb)
                        if j == GN - 1:
                            wait_gidx(1 - gb)
                            start_rows(k + 1, 1 - gb, 0, nrb)
                        else:
                            start_rows(k + 1, gb, j + 1, nrb)

                    def wbody(i, _):
                        sl = pl.ds(i * 16, 16)
                        d16 = idxd_g[gb][j, sl]
                        pd = plsc.load_gather(p_v, [d16])
                        if gather_rows:
                            qq = plsc.load_gather(
                                ps_v, [idxs_g[gb][j, sl] - c * NP])
                        else:
                            qq = qg_b[gb][j, sl]
                        lo = pd + qq
                        lo = jnp.where(lo >= 0, lo, 0.01 * lo)
                        w16 = jnp.exp(lo)
                        wv[sl] = w16
                        plsc.addupdate_scatter(s_v, [d16], w16)
                        return 0

                    lax.fori_loop(0, kc // 16, wbody, 0)
                    _scale_loop(rows_b[rb], wv, kc, 128)
                    pltpu.async_copy(rows_b[rb], acc_sh.at[idxd_g[gb].at[j]],
                                     csem[rb], add=True)
            return 0

        lax.fori_loop(0, ngrp // 2, gpair, 0)
        wait_scat(0)
        wait_scat(1)
        pltpu.sync_copy(s_v, outSP.at[c].at[t])
        plsc.subcore_barrier()
        pltpu.sync_copy(acc_sh.at[pl.ds(t * NPT, NPT)],
                        outC.at[c].at[pl.ds(t * NPT, NPT)])

        def redk(kk, _):
            off = t * NPT + kk * 128
            pltpu.sync_copy(outSP.at[c].at[:, pl.ds(off, 128)],
                            rows0.at[pl.ds(0, NS)])
            for ff in range(8):
                sl = pl.ds(ff * 16, 16)
                a = rows0[0, sl]
                for r in range(1, NS):
                    a = a + rows0[r, sl]
                rows0[NS, sl] = a
            pltpu.sync_copy(rows0.at[NS], outS.at[c].at[pl.ds(off, 128)])
            return 0

        lax.fori_loop(0, NPT // 128, redk, 0)

    f = pl.kernel(
        body,
        out_type=(jax.ShapeDtypeStruct((2, NP, 128), jnp.float32),
                  jax.ShapeDtypeStruct((2, NP), jnp.float32),
                  jax.ShapeDtypeStruct((2, NS, NP), jnp.float32)),
        mesh=_get_mesh(),
        compiler_params=pltpu.CompilerParams(needs_layout_passes=False),
        scratch_types=[
            pltpu.VMEM((PV,), jnp.float32),
            pltpu.VMEM((PV,) if gather_rows else (16,), jnp.float32),
            pltpu.VMEM((GN, kc), jnp.int32),
            pltpu.VMEM((GN, kc), jnp.int32),
            pltpu.VMEM((GN, kc) if gather_rows else (1, 16), jnp.int32),
            pltpu.VMEM((GN, kc) if gather_rows else (1, 16), jnp.int32),
            pltpu.VMEM((kc, 128), jnp.float32),
            pltpu.VMEM((kc, 128), jnp.float32),
            pltpu.VMEM((1, 16) if gather_rows else (GN, kc), jnp.float32),
            pltpu.VMEM((1, 16) if gather_rows else (GN, kc), jnp.float32),
            pltpu.VMEM((kc,), jnp.float32),
            pltpu.VMEM((NP,), jnp.float32),
            pltpu.VMEM_SHARED((NP, 128), jnp.float32),
            pltpu.SemaphoreType.DMA,
            pltpu.SemaphoreType.DMA,
            pltpu.SemaphoreType.DMA,
            pltpu.SemaphoreType.DMA,
            pltpu.SemaphoreType.DMA,
            pltpu.SemaphoreType.DMA,
        ],
    )
    return f


def _sck_nodes(scale):
    def body(rows_h, tg_h, nb_h, gid_h, z128_h, z1_h, *rest):
        if scale:
            (outC, outS, outSP, tg_v, idx_v, rows_v, qv, wv, s_v, sp_v,
             sred_v, acc_sh) = rest
        else:
            (outC, idx_v, rows_v, acc_sh) = rest
        c = lax.axis_index("c")
        t = lax.axis_index("s")
        zr = NB // NS
        pltpu.sync_copy(z128_h.at[pl.ds(0, zr)], acc_sh.at[pl.ds(t * zr, zr)])
        if scale:
            pltpu.sync_copy(z1_h.at[pl.ds(0, NB)], s_v)
            pltpu.sync_copy(tg_h.at[c], tg_v)
        plsc.subcore_barrier()

        nbase = t * NPT
        fbase = c * NP + nbase
        pltpu.sync_copy(gid_h.at[c].at[pl.ds(nbase, NPT)], idx_v)
        pltpu.sync_copy(rows_h.at[pl.ds(fbase, NPT)], rows_v)
        if scale:
            pltpu.sync_copy(nb_h.at[pl.ds(fbase, NPT)], qv)

            def wbody(i, _):
                sl = pl.ds(i * 16, 16)
                d16 = idx_v[sl]
                pd = plsc.load_gather(tg_v, [d16])
                lo = pd + qv[sl]
                lo = jnp.where(lo >= 0, lo, 0.01 * lo)
                w16 = jnp.exp(lo)
                wv[sl] = w16
                plsc.addupdate_scatter(s_v, [d16], w16)
                return 0

            lax.fori_loop(0, NPT // 16, wbody, 0)
            _scale_loop(rows_v, wv, NPT, 128)
        pltpu.sync_copy(rows_v, acc_sh.at[idx_v], add=True)
        if scale:
            pltpu.sync_copy(s_v, outSP.at[c].at[t])
        plsc.subcore_barrier()
        pltpu.sync_copy(acc_sh.at[pl.ds(t * zr, zr)],
                        outC.at[c].at[pl.ds(t * zr, zr)])
        if scale:
            @pl.when(t == 0)
            def _reduce():
                pltpu.sync_copy(outSP.at[c], sp_v)

                def redk(i, _):
                    sl = pl.ds(i * 16, 16)
                    a = sp_v[0, sl]
                    for r in range(1, NS):
                        a = a + sp_v[r, sl]
                    sred_v[sl] = a
                    return 0

                lax.fori_loop(0, NB // 16, redk, 0)
                pltpu.sync_copy(sred_v, outS.at[c])

    if scale:
        out_type = (jax.ShapeDtypeStruct((2, NB, 128), jnp.float32),
                    jax.ShapeDtypeStruct((2, NB), jnp.float32),
                    jax.ShapeDtypeStruct((2, NS, NB), jnp.float32))
        scratch = [
            pltpu.VMEM((NB,), jnp.float32),
            pltpu.VMEM((NPT,), jnp.int32),
            pltpu.VMEM((NPT, 128), jnp.float32),
            pltpu.VMEM((NPT,), jnp.float32),
            pltpu.VMEM((NPT,), jnp.float32),
            pltpu.VMEM((NB,), jnp.float32),
            pltpu.VMEM((NS, NB), jnp.float32),
            pltpu.VMEM((NB,), jnp.float32),
            pltpu.VMEM_SHARED((NB, 128), jnp.float32),
        ]
    else:
        out_type = jax.ShapeDtypeStruct((2, NB, 128), jnp.float32)
        scratch = [
            pltpu.VMEM((NPT,), jnp.int32),
            pltpu.VMEM((NPT, 128), jnp.float32),
            pltpu.VMEM_SHARED((NB, 128), jnp.float32),
        ]
    return pl.kernel(body, out_type=out_type, mesh=_get_mesh(),
        compiler_params=pltpu.CompilerParams(needs_layout_passes=False),
                     scratch_types=scratch)


def _tck_pre(x, Wpn, bpn, u, W1a, b1):
    def body(x_r, W_r, b_r, u_r, Wa_r, b1_r, hv_r, p_r, xa_r):
        xb = x_r[...]
        h = _leaky(jnp.dot(xb, W_r[...],
                           preferred_element_type=jnp.float32) + b_r[...])
        hv_r[...] = h
        p_r[...] = jnp.dot(h, u_r[...], preferred_element_type=jnp.float32)
        xa_r[...] = jnp.dot(xb, Wa_r[...],
                            preferred_element_type=jnp.float32) + b1_r[...]

    return pl.pallas_call(
        body,
        grid=(SN // RB,),
        in_specs=[
            pl.BlockSpec((RB, 32), lambda i: (i, 0)),
            pl.BlockSpec((32, 128), lambda i: (0, 0)),
            pl.BlockSpec((1, 128), lambda i: (0, 0)),
            pl.BlockSpec((128, 1), lambda i: (0, 0)),
            pl.BlockSpec((32, 128), lambda i: (0, 0)),
            pl.BlockSpec((1, 128), lambda i: (0, 0)),
        ],
        out_specs=[
            pl.BlockSpec((RB, 128), lambda i: (i, 0)),
            pl.BlockSpec((RB, 1), lambda i: (i, 0)),
            pl.BlockSpec((RB, 128), lambda i: (i, 0)),
        ],
        out_shape=[
            jax.ShapeDtypeStruct((SN, 128), jnp.float32),
            jax.ShapeDtypeStruct((SN, 1), jnp.float32),
            jax.ShapeDtypeStruct((SN, 128), jnp.float32),
        ],
    )(x, Wpn, bpn, u, W1a, b1)


def _tck_edge_mlp(xa_src, eT, Wb, v, b2):
    def body(xa_r, e_r, Wb_r, v_r, b2_r, he_r, q_r):
        eb = lax.dot_general(e_r[...], Wb_r[...],
                             (((0,), (0,)), ((), ())),
                             preferred_element_type=jnp.float32)
        h = _leaky(xa_r[...] + eb)
        he_r[...] = h
        q2 = jnp.dot(h, v_r[...],
                     preferred_element_type=jnp.float32) + b2_r[...]
        q_r[...] = q2.reshape(REB // KC, KC)

    return pl.pallas_call(
        body,
        grid=(SE // REB,),
        in_specs=[
            pl.BlockSpec((REB, 128), lambda i: (i, 0)),
            pl.BlockSpec((6, REB), lambda i: (0, i)),
            pl.BlockSpec((6, 128), lambda i: (0, 0)),
            pl.BlockSpec((128, 1), lambda i: (0, 0)),
            pl.BlockSpec((1, 1), lambda i: (0, 0)),
        ],
        out_specs=[
            pl.BlockSpec((REB, 128), lambda i: (i, 0)),
            pl.BlockSpec((REB // KC, KC), lambda i: (i, 0)),
        ],
        out_shape=[
            jax.ShapeDtypeStruct((SE, 128), jnp.float32),
            jax.ShapeDtypeStruct((SE // KC, KC), jnp.float32),
        ],
    )(xa_src, eT, Wb, v, b2)


def _tck_post_gc(Cw, sw, hvnew, Wet, bet, WihT, WhhT, bih, bhh,
                 u1, u2, bu2, Wpnode, bpnode):
    def body(Cw_r, s_r, hv_r, Wet_r, bet_r, WihT_r, WhhT_r, bih_r, bhh_r,
             u1_r, u2_r, bu2_r, Wpn_r, bpn_r,
             node_r, hvo_r, p1_r, p2_r):
        s = s_r[...]
        inv = 1.0 / (s + EPS)
        sn = s * inv
        c = jnp.dot(Cw_r[...], Wet_r[...],
                    preferred_element_type=jnp.float32) * inv + bet_r[...] * sn
        node = jax.nn.relu(_gru_tc(_elu(c), hv_r[...], WihT_r[...],
                                   WhhT_r[...], bih_r[...], bhh_r[...]))
        node_r[...] = node
        hvo_r[...] = jnp.dot(node, Wpn_r[...],
                             preferred_element_type=jnp.float32) + bpn_r[...]
        p1_r[...] = jnp.dot(node, u1_r[...], preferred_element_type=jnp.float32)
        p2_r[...] = jnp.dot(node, u2_r[...],
                            preferred_element_type=jnp.float32) + bu2_r[...]

    return pl.pallas_call(
        body,
        grid=(SN // RB,),
        in_specs=[
            pl.BlockSpec((RB, 128), lambda i: (i, 0)),
            pl.BlockSpec((RB, 1), lambda i: (i, 0)),
            pl.BlockSpec((RB, 128), lambda i: (i, 0)),
            pl.BlockSpec((128, 128), lambda i: (0, 0)),
            pl.BlockSpec((1, 128), lambda i: (0, 0)),
            pl.BlockSpec((128, 384), lambda i: (0, 0)),
            pl.BlockSpec((128, 384), lambda i: (0, 0)),
            pl.BlockSpec((1, 384), lambda i: (0, 0)),
            pl.BlockSpec((1, 384), lambda i: (0, 0)),
            pl.BlockSpec((128, 1), lambda i: (0, 0)),
            pl.BlockSpec((128, 1), lambda i: (0, 0)),
            pl.BlockSpec((1, 1), lambda i: (0, 0)),
            pl.BlockSpec((128, 128), lambda i: (0, 0)),
            pl.BlockSpec((1, 128), lambda i: (0, 0)),
        ],
        out_specs=[
            pl.BlockSpec((RB, 128), lambda i: (i, 0)),
            pl.BlockSpec((RB, 128), lambda i: (i, 0)),
            pl.BlockSpec((RB, 1), lambda i: (i, 0)),
            pl.BlockSpec((RB, 1), lambda i: (i, 0)),
        ],
        out_shape=[
            jax.ShapeDtypeStruct((SN, 128), jnp.float32),
            jax.ShapeDtypeStruct((SN, 128), jnp.float32),
            jax.ShapeDtypeStruct((SN, 1), jnp.float32),
            jax.ShapeDtypeStruct((SN, 1), jnp.float32),
        ],
    )(Cw, sw, hvnew, Wet, bet, WihT, WhhT, bih, bhh, u1, u2, bu2,
      Wpnode, bpnode)


def _tck_post_l1(C2, s2, node, WihT, WhhT, bih, bhh,
                 Wpn0, bpn0, Wpn1, bpn1, wb0, bcl0, wb1, bcl1):
    def body(C_r, s_r, nd_r, WihT_r, WhhT_r, bih_r, bhh_r,
             Wpn0_r, bpn0_r, Wpn1_r, bpn1_r, wb0_r, bcl0_r, wb1_r, bcl1_r,
             n2_r, hv0_r, hv1_r, nb0_r, nb1_r):
        c = C_r[...] / (s_r[...] + EPS)
        n2 = jax.nn.relu(_gru_tc(_elu(c), nd_r[...], WihT_r[...],
                                 WhhT_r[...], bih_r[...], bhh_r[...]))
        n2_r[...] = n2
        hv0_r[...] = jnp.dot(n2, Wpn0_r[...],
                             preferred_element_type=jnp.float32) + bpn0_r[...]
        hv1_r[...] = jnp.dot(n2, Wpn1_r[...],
                             preferred_element_type=jnp.float32) + bpn1_r[...]
        nb0_r[...] = jnp.dot(n2, wb0_r[...],
                             preferred_element_type=jnp.float32) + bcl0_r[...]
        nb1_r[...] = jnp.dot(n2, wb1_r[...],
                             preferred_element_type=jnp.float32) + bcl1_r[...]

    return pl.pallas_call(
        body,
        grid=(SN // RB,),
        in_specs=[
            pl.BlockSpec((RB, 128), lambda i: (i, 0)),
            pl.BlockSpec((RB, 1), lambda i: (i, 0)),
            pl.BlockSpec((RB, 128), lambda i: (i, 0)),
            pl.BlockSpec((128, 384), lambda i: (0, 0)),
            pl.BlockSpec((128, 384), lambda i: (0, 0)),
            pl.BlockSpec((1, 384), lambda i: (0, 0)),
            pl.BlockSpec((1, 384), lambda i: (0, 0)),
            pl.BlockSpec((128, 128), lambda i: (0, 0)),
            pl.BlockSpec((1, 128), lambda i: (0, 0)),
            pl.BlockSpec((128, 128), lambda i: (0, 0)),
            pl.BlockSpec((1, 128), lambda i: (0, 0)),
            pl.BlockSpec((128, 1), lambda i: (0, 0)),
            pl.BlockSpec((1, 1), lambda i: (0, 0)),
            pl.BlockSpec((128, 1), lambda i: (0, 0)),
            pl.BlockSpec((1, 1), lambda i: (0, 0)),
        ],
        out_specs=[pl.BlockSpec((RB, 128), lambda i: (i, 0))] * 3
        + [pl.BlockSpec((RB, 1), lambda i: (i, 0))] * 2,
        out_shape=[jax.ShapeDtypeStruct((SN, 128), jnp.float32)] * 3
        + [jax.ShapeDtypeStruct((SN, 1), jnp.float32)] * 2,
    )(C2, s2, node, WihT, WhhT, bih, bhh, Wpn0, bpn0, Wpn1, bpn1,
      wb0, bcl0, wb1, bcl1)


def _tck_tg(G, wa):
    def body(G_r, wa_r, tg_r):
        tg_r[...] = jnp.dot(jax.nn.relu(G_r[...]), wa_r[...],
                            preferred_element_type=jnp.float32)

    return pl.pallas_call(
        body,
        grid=(1,),
        in_specs=[
            pl.BlockSpec((2 * NB, 128), lambda i: (0, 0)),
            pl.BlockSpec((128, 1), lambda i: (0, 0)),
        ],
        out_specs=pl.BlockSpec((2 * NB, 1), lambda i: (0, 0)),
        out_shape=jax.ShapeDtypeStruct((2 * NB, 1), jnp.float32),
    )(G, wa)


def _tck_ro_gru(G, s, h, WihT, WhhT, bih, bhh, wa_next):
    def body(G_r, s_r, h_r, WihT_r, WhhT_r, bih_r, bhh_r, wa_r, g_r, tg_r):
        g_repr = _elu(G_r[...] / (s_r[...] + EPS))
        g = jax.nn.relu(_gru_tc(g_repr, h_r[...], WihT_r[...], WhhT_r[...],
                                bih_r[...], bhh_r[...]))
        g_r[...] = g
        tg_r[...] = jnp.dot(jax.nn.relu(g), wa_r[...],
                            preferred_element_type=jnp.float32)

    return pl.pallas_call(
        body,
        grid=(1,),
        in_specs=[
            pl.BlockSpec((2 * NB, 128), lambda i: (0, 0)),
            pl.BlockSpec((2 * NB, 1), lambda i: (0, 0)),
            pl.BlockSpec((2 * NB, 128), lambda i: (0, 0)),
            pl.BlockSpec((128, 384), lambda i: (0, 0)),
            pl.BlockSpec((128, 384), lambda i: (0, 0)),
            pl.BlockSpec((1, 384), lambda i: (0, 0)),
            pl.BlockSpec((1, 384), lambda i: (0, 0)),
            pl.BlockSpec((128, 1), lambda i: (0, 0)),
        ],
        out_specs=[
            pl.BlockSpec((2 * NB, 128), lambda i: (0, 0)),
            pl.BlockSpec((2 * NB, 1), lambda i: (0, 0)),
        ],
        out_shape=[
            jax.ShapeDtypeStruct((2 * NB, 128), jnp.float32),
            jax.ShapeDtypeStruct((2 * NB, 1), jnp.float32),
        ],
    )(G, s, h, WihT, WhhT, bih, bhh, wa_next)


def _tck_final(G, s, h, WihT, WhhT, bih, bhh, Wpred, bpred,
               WfcA, WfcB, bfc, bn_a, bn_b, Wout, bout):
    def body(G_r, s_r, h_r, WihT_r, WhhT_r, bih_r, bhh_r, Wp_r, bp_r,
             WA_r, WB_r, bfc_r, bna_r, bnb_r, Wo_r, bo_r, o_r):
        g_repr = _elu(G_r[...] / (s_r[...] + EPS))
        g = jax.nn.relu(_gru_tc(g_repr, h_r[...], WihT_r[...], WhhT_r[...],
                                bih_r[...], bhh_r[...]))
        pred = jnp.dot(g, Wp_r[...],
                       preferred_element_type=jnp.float32) + bp_r[...]
        s1 = pred[0:B, :]
        s2 = pred[NB:NB + B, :]
        hh = (jnp.dot(s1, WA_r[...], preferred_element_type=jnp.float32)
              + jnp.dot(s2, WB_r[...], preferred_element_type=jnp.float32)
              + bfc_r[...])
        hh = jax.nn.relu(hh * bna_r[...] + bnb_r[...])
        o_r[...] = jnp.dot(hh, Wo_r[...],
                           preferred_element_type=jnp.float32) + bo_r[...]

    return pl.pallas_call(
        body,
        grid=(1,),
        in_specs=[
            pl.BlockSpec((2 * NB, 128), lambda i: (0, 0)),
            pl.BlockSpec((2 * NB, 1), lambda i: (0, 0)),
            pl.BlockSpec((2 * NB, 128), lambda i: (0, 0)),
            pl.BlockSpec((128, 384), lambda i: (0, 0)),
            pl.BlockSpec((128, 384), lambda i: (0, 0)),
            pl.BlockSpec((1, 384), lambda i: (0, 0)),
            pl.BlockSpec((1, 384), lambda i: (0, 0)),
            pl.BlockSpec((128, 256), lambda i: (0, 0)),
            pl.BlockSpec((1, 256), lambda i: (0, 0)),
            pl.BlockSpec((256, 1024), lambda i: (0, 0)),
            pl.BlockSpec((256, 1024), lambda i: (0, 0)),
            pl.BlockSpec((1, 1024), lambda i: (0, 0)),
            pl.BlockSpec((1, 1024), lambda i: (0, 0)),
            pl.BlockSpec((1, 1024), lambda i: (0, 0)),
            pl.BlockSpec((1024, 1), lambda i: (0, 0)),
            pl.BlockSpec((1, 1), lambda i: (0, 0)),
        ],
        out_specs=pl.BlockSpec((B, 1), lambda i: (0, 0)),
        out_shape=jax.ShapeDtypeStruct((B, 1), jnp.float32),
    )(G, s, h, WihT, WhhT, bih, bhh, Wpred, bpred, WfcA, WfcB, bfc,
      bn_a, bn_b, Wout, bout)


def kernel(x1, e1, edge_index1, gid1, x2, e2, edge_index2, gid2, Wfc, Wout,
           Wpred, bfc, bn_beta, bn_gamma, bn_mean, bn_var, bout, bpred,
           gc_Wet, gc_Whh, gc_Wih, gc_Wpe1, gc_Wpe2, gc_Wpn, gc_bet, gc_bhh,
           gc_bih, gc_bpe1, gc_bpe2, gc_bpn, l1_Whh, l1_Wih, l1_Wpe,
           l1_Wpnode, l1_bhh, l1_bih, l1_bpe, l1_bpnode, ro0_Wcl, ro0_Whh,
           ro0_Wih, ro0_Wpn, ro0_bcl, ro0_bhh, ro0_bih, ro0_bpn, ro1_Wcl,
           ro1_Whh, ro1_Wih, ro1_Wpn, ro1_bcl, ro1_bhh, ro1_bih, ro1_bpn):
    f32 = jnp.float32
    pad_n = NP - N
    pad_e = EP - E
    xs_pad = lambda a: jnp.pad(a, ((0, pad_n), (0, 0)))
    ep2 = lambda a: jnp.pad(a, ((0, pad_e), (0, 0)))
    ep1 = lambda a, v=0: jnp.pad(a, (0, pad_e), constant_values=v)
    x2n = jnp.concatenate([xs_pad(x1), xs_pad(x2)], axis=0)
    eT = jnp.concatenate([ep2(e1), ep2(e2)], axis=0).T
    src_b = jnp.concatenate([ep1(edge_index1[0]),
                             ep1(edge_index2[0]) + NP])
    dst2d = jnp.stack([ep1(edge_index1[1], N),
                       ep1(edge_index2[1], N)])
    gid_pad = lambda g: jnp.pad(g, (0, pad_n), constant_values=B)
    gid2d = jnp.stack([gid_pad(gid1), gid_pad(gid2)])
    z128 = jnp.zeros((NP, 128), f32)
    z1 = jnp.zeros((NP,), f32)

    row = lambda b: b.reshape(1, -1)
    col = lambda w: w.reshape(-1, 1)
    u_gc = col(gc_Wpe2[:128, 0])
    v_gc = col(gc_Wpe2[128:, 0])
    u1_l1 = col(l1_Wpe[:128, 0])
    u2_l1 = col(l1_Wpe[128:, 0])
    wa0, wb0 = col(ro0_Wcl[:128, 0]), col(ro0_Wcl[128:, 0])
    wa1, wb1 = col(ro1_Wcl[:128, 0]), col(ro1_Wcl[128:, 0])
    bn_a = row(bn_gamma / jnp.sqrt(bn_var + 1e-5))
    bn_b = row(bn_beta - bn_mean * bn_gamma / jnp.sqrt(bn_var + 1e-5))
    WfcA, WfcB = Wfc[:256], Wfc[256:]

    hv_new, p_gc, xa = _tck_pre(x2n, gc_Wpn, row(gc_bpn), u_gc,
                                gc_Wpe1[:32], row(gc_bpe1))
    xa_src = _sck_gather_rows(xa, src_b, 128)
    he1, q_gc = _tck_edge_mlp(xa_src, eT, gc_Wpe1[32:],
                              v_gc, gc_bpe2.reshape(1, 1))
    p2d_gc = p_gc.reshape(2, NP)
    edge_gc = _sck_edge(gather_rows=False)
    dst3 = dst2d.reshape(2, EP // KC, KC)
    src3 = src_b.reshape(SE // KC, KC)
    Cw, sw, _sp1 = edge_gc(he1, q_gc, p2d_gc, dst3,
                           src3, z128, z1)
    node, hv_l1, p1, p2 = _tck_post_gc(
        Cw.reshape(SN, 128), sw.reshape(SN, 1), hv_new,
        gc_Wet, row(gc_bet), gc_Wih.T, gc_Whh.T, row(gc_bih), row(gc_bhh),
        u1_l1, u2_l1, l1_bpe.reshape(1, 1), l1_Wpnode, row(l1_bpnode))

    edge_l1 = _sck_edge(gather_rows=True)
    dst3b = dst2d.reshape(2, EP // 32, 32)
    src3b = src_b.reshape(SE // 32, 32)
    C2, s2, _sp2 = edge_l1(hv_l1, p2.reshape(2, NP), p1.reshape(2, NP),
                           dst3b, src3b, z128, z1)
    node2, hv0, hv1, nb0, nb1 = _tck_post_l1(
        C2.reshape(SN, 128), s2.reshape(SN, 1), node,
        l1_Wih.T, l1_Whh.T, row(l1_bih), row(l1_bhh),
        ro0_Wpn, row(ro0_bpn), ro1_Wpn, row(ro1_bpn),
        wb0, ro0_bcl.reshape(1, 1), wb1, ro1_bcl.reshape(1, 1))

    zt = jnp.zeros((2, NB), f32)
    zn = jnp.zeros((SN,), f32)
    gf = _sck_nodes(scale=False)(node2, zt, zn, gid2d, z128, z1)
    gfeats = gf.reshape(2 * NB, 128)
    tg0 = _tck_tg(gfeats, wa0)
    ro_k = _sck_nodes(scale=True)
    G0, S0, _sp3 = ro_k(hv0, tg0.reshape(2, NB), nb0.reshape(SN), gid2d, z128, z1)
    gf1, tg1 = _tck_ro_gru(G0.reshape(2 * NB, 128),
                           S0.reshape(2 * NB, 1), gfeats,
                           ro0_Wih.T, ro0_Whh.T, row(ro0_bih), row(ro0_bhh),
                           wa1)
    G1, S1, _sp4 = ro_k(hv1, tg1.reshape(2, NB), nb1.reshape(SN), gid2d, z128, z1)
    o = _tck_final(G1.reshape(2 * NB, 128), S1.reshape(2 * NB, 1),
                   gf1, ro1_Wih.T, ro1_Whh.T, row(ro1_bih), row(ro1_bhh),
                   Wpred, row(bpred), WfcA, WfcB, row(bfc), bn_a, bn_b,
                   Wout, bout.reshape(1, 1))
    return o.reshape(B)


# speedup vs baseline: 1.2576x; 1.2576x over previous
import functools

import jax
import jax.numpy as jnp
from jax import lax
from jax.experimental import pallas as pl
from jax.experimental.pallas import tpu as pltpu
from jax.experimental.pallas import tpu_sc as plsc

N, E, B = 10000, 160000, 512
NP = 10240
SN = 2 * NP
EP = 163840
SE = 2 * EP
NC, NS = 2, 16
EPT = EP // NS
KC = 64
NPT = NP // NS
NB = 640
EPS = 1e-9
RB = 2048
REB = 2048

@functools.cache
def _get_mesh():
    return plsc.VectorSubcoreMesh(core_axis_name="c", subcore_axis_name="s",
                                  num_cores=NC, num_subcores=NS)


def _leaky(x):
    return jnp.where(x >= 0, x, 0.01 * x)


def _elu(x):
    return jnp.where(x > 0, x, jnp.exp(x) - 1.0)


def _gru_tc(x, h, wihT, whhT, bih, bhh):
    gi = jnp.dot(x, wihT, preferred_element_type=jnp.float32) + bih
    gh = jnp.dot(h, whhT, preferred_element_type=jnp.float32) + bhh
    r = jax.nn.sigmoid(gi[:, 0:128] + gh[:, 0:128])
    z = jax.nn.sigmoid(gi[:, 128:256] + gh[:, 128:256])
    n = jnp.tanh(gi[:, 256:384] + r * gh[:, 256:384])
    return (1.0 - z) * n + z * h


def _sck_gather_rows(table3, idx, d, dtype=jnp.float32):
    kc = 128
    nch = EPT // kc

    def body(tab_h, idx_h, out_h, idx0, idx1, rows0, rows1, tab_sh,
             g0, g1, s0, s1):
        c = lax.axis_index("c")
        t = lax.axis_index("s")
        idx_b = [idx0, idx1]
        rows_b = [rows0, rows1]
        gsem = [g0, g1]
        ssem = [s0, s1]
        pltpu.sync_copy(tab_h.at[c].at[pl.ds(t * NPT, NPT)],
                        tab_sh.at[pl.ds(t * NPT, NPT)])
        plsc.subcore_barrier()
        base = c * EP + t * EPT

        def start_gather(k, b):
            pltpu.sync_copy(idx_h.at[pl.ds(base + k * kc, kc)], idx_b[b])

            def adj(i, _):
                sl = pl.ds(i * 16, 16)
                idx_b[b][sl] = idx_b[b][sl] - c * NP
                return 0

            lax.fori_loop(0, kc // 16, adj, 0)
            pltpu.async_copy(tab_sh.at[idx_b[b]], rows_b[b], gsem[b])

        start_gather(0, 0)

        def pair(g, _):
            for b in range(2):
                k = g * 2 + b
                nb = 1 - b
                pltpu.make_async_copy(tab_sh.at[idx_b[b]], rows_b[b],
                                      gsem[b]).wait()

                @pl.when(k + 1 < nch)
                def _pre():
                    @pl.when(k >= 1)
                    def _drain():
                        pltpu.make_async_copy(
                            rows_b[nb], out_h.at[pl.ds(base, kc)],
                            ssem[nb]).wait()
                    start_gather(k + 1, nb)

                pltpu.async_copy(rows_b[b], out_h.at[pl.ds(base + k * kc, kc)],
                                 ssem[b])
            return 0

        lax.fori_loop(0, nch // 2, pair, 0)
        pltpu.make_async_copy(rows_b[0], out_h.at[pl.ds(base, kc)],
                              ssem[0]).wait()
        pltpu.make_async_copy(rows_b[1], out_h.at[pl.ds(base, kc)],
                              ssem[1]).wait()

    f = pl.kernel(
        body,
        out_type=jax.ShapeDtypeStruct((SE, d), dtype),
        mesh=_get_mesh(),
        compiler_params=pltpu.CompilerParams(needs_layout_passes=False),
        scratch_types=[
            pltpu.VMEM((kc,), jnp.int32),
            pltpu.VMEM((kc,), jnp.int32),
            pltpu.VMEM((kc, d), dtype),
            pltpu.VMEM((kc, d), dtype),
            pltpu.VMEM_SHARED((NP, d), dtype),
            pltpu.SemaphoreType.DMA,
            pltpu.SemaphoreType.DMA,
            pltpu.SemaphoreType.DMA,
            pltpu.SemaphoreType.DMA,
        ],
    )
    return f(table3, idx)


def _scale_loop(rows_v, wv, kc, dcols):
    def sbody(j, _):
        wj = plsc.load_gather(wv, [jnp.full((16,), j, jnp.int32)])
        for f in range(dcols // 16):
            sl = pl.ds(f * 16, 16)
            rows_v[j, sl] = rows_v[j, sl] * wj
        return 0

    lax.fori_loop(0, kc, sbody, 0)


GN = 4
PV = 10112


def _sck_edge(gather_rows):
    kc = 32 if gather_rows else KC
    nch = EPT // kc
    ngrp = nch // GN
    CPT = EPT // kc

    def body(rows_h, q_h, p_h, dst_h, src_h, z128_h, z1_h, outC, outS, outSP,
             p_v, ps_v, idxd0, idxd1, idxs0, idxs1, rows0, rows1, qg0, qg1,
             wv, s_v, acc_sh, i0, i1, g0, g1, c0, c1):
        c = lax.axis_index("c")
        t = lax.axis_index("s")
        idxd_g = [idxd0, idxd1]
        idxs_g = [idxs0, idxs1]
        rows_b = [rows0, rows1]
        qg_b = [qg0, qg1]
        isem = [i0, i1]
        gsem = [g0, g1]
        csem = [c0, c1]
        pltpu.sync_copy(z128_h.at[pl.ds(t * NPT, NPT)],
                        acc_sh.at[pl.ds(t * NPT, NPT)])
        pltpu.sync_copy(z1_h, s_v)
        pltpu.sync_copy(p_h.at[c].at[pl.ds(0, PV)], p_v)
        if gather_rows:
            pltpu.sync_copy(q_h.at[c].at[pl.ds(0, PV)], ps_v)
        plsc.subcore_barrier()

        def g_idx_load(m, buf):
            crow = t * CPT + m * GN
            frow = c * CPT * NS + crow
            pltpu.async_copy(dst_h.at[c].at[pl.ds(crow, GN)], idxd_g[buf],
                             isem[buf])
            if gather_rows:
                pltpu.async_copy(src_h.at[pl.ds(frow, GN)], idxs_g[buf],
                                 isem[buf])
            else:
                pltpu.async_copy(q_h.at[pl.ds(frow, GN)], qg_b[buf],
                                 isem[buf])

        def wait_gidx(buf):
            pltpu.make_async_copy(dst_h.at[c].at[pl.ds(0, GN)], idxd_g[buf],
                                  isem[buf]).wait()
            if gather_rows:
                pltpu.make_async_copy(src_h.at[pl.ds(0, GN)], idxs_g[buf],
                                      isem[buf]).wait()
            else:
                pltpu.make_async_copy(q_h.at[pl.ds(0, GN)], qg_b[buf],
                                      isem[buf]).wait()

        def start_rows(k, gb, j, rb):
            if gather_rows:
                pltpu.async_copy(rows_h.at[idxs_g[gb].at[j]], rows_b[rb],
                                 gsem[rb])
            else:
                fbase = c * EP + t * EPT + k * kc
                pltpu.async_copy(rows_h.at[pl.ds(fbase, KC)], rows_b[rb],
                                 gsem[rb])

        def wait_rows(rb):
            if gather_rows:
                pltpu.make_async_copy(rows_h.at[idxs_g[0].at[0]], rows_b[rb],
                                      gsem[rb]).wait()
            else:
                pltpu.make_async_copy(rows_h.at[pl.ds(0, kc)], rows_b[rb],
                                      gsem[rb]).wait()

        def wait_scat(rb):
            pltpu.make_async_copy(rows_b[rb], acc_sh.at[idxd_g[0].at[0]],
                                  csem[rb]).wait()

        g_idx_load(0, 0)
        wait_gidx(0)
        start_rows(0, 0, 0, 0)
        g_idx_load(1, 1)

        def gpair(gp, _):
            for gpar in range(2):
                g = gp * 2 + gpar
                gb = gpar
                for j in range(GN):
                    k = g * GN + j
                    rb = j % 2
                    nrb = 1 - rb
                    wait_rows(rb)
                    if j == 1:
                        @pl.when(jnp.logical_and(g >= 1, g + 1 < ngrp))
                        def _ld():
                            g_idx_load(g + 1, 1 - gb)

                    @pl.when(k + 1 < nch)
                    def _pre():
                        @pl.when(k >= 1)
                        def _drain():
                            wait_scat(nrb)
                        if j == GN - 1:
                            wait_gidx(1 - gb)
                            start_rows(k + 1, 1 - gb, 0, nrb)
                        else:
                            start_rows(k + 1, gb, j + 1, nrb)

                    def wbody(i, _):
                        sl = pl.ds(i * 16, 16)
                        d16 = idxd_g[gb][j, sl]
                        pd = plsc.load_gather(p_v, [d16])
                        if gather_rows:
                            qq = plsc.load_gather(
                                ps_v, [idxs_g[gb][j, sl] - c * NP])
                        else:
                            qq = qg_b[gb][j, sl]
                        lo = pd + qq
                        lo = jnp.where(lo >= 0, lo, 0.01 * lo)
                        w16 = jnp.exp(lo)
                        wv[sl] = w16
                        plsc.addupdate_scatter(s_v, [d16], w16)
                        return 0

                    lax.fori_loop(0, kc // 16, wbody, 0)
                    _scale_loop(rows_b[rb], wv, kc, 128)
                    pltpu.async_copy(rows_b[rb], acc_sh.at[idxd_g[gb].at[j]],
                                     csem[rb], add=True)
            return 0

        lax.fori_loop(0, ngrp // 2, gpair, 0)
        wait_scat(0)
        wait_scat(1)
        pltpu.sync_copy(s_v, outSP.at[c].at[t])
        plsc.subcore_barrier()
        pltpu.sync_copy(acc_sh.at[pl.ds(t * NPT, NPT)],
                        outC.at[c].at[pl.ds(t * NPT, NPT)])

        def redk(kk, _):
            off = t * NPT + kk * 128
            pltpu.sync_copy(outSP.at[c].at[:, pl.ds(off, 128)],
                            rows0.at[pl.ds(0, NS)])
            for ff in range(8):
                sl = pl.ds(ff * 16, 16)
                a = rows0[0, sl]
                for r in range(1, NS):
                    a = a + rows0[r, sl]
                rows0[NS, sl] = a
            pltpu.sync_copy(rows0.at[NS], outS.at[c].at[pl.ds(off, 128)])
            return 0

        lax.fori_loop(0, NPT // 128, redk, 0)

    f = pl.kernel(
        body,
        out_type=(jax.ShapeDtypeStruct((2, NP, 128), jnp.float32),
                  jax.ShapeDtypeStruct((2, NP), jnp.float32),
                  jax.ShapeDtypeStruct((2, NS, NP), jnp.float32)),
        mesh=_get_mesh(),
        compiler_params=pltpu.CompilerParams(needs_layout_passes=False),
        scratch_types=[
            pltpu.VMEM((PV,), jnp.float32),
            pltpu.VMEM((PV,) if gather_rows else (16,), jnp.float32),
            pltpu.VMEM((GN, kc), jnp.int32),
            pltpu.VMEM((GN, kc), jnp.int32),
            pltpu.VMEM((GN, kc) if gather_rows else (1, 16), jnp.int32),
            pltpu.VMEM((GN, kc) if gather_rows else (1, 16), jnp.int32),
            pltpu.VMEM((kc, 128), jnp.float32),
            pltpu.VMEM((kc, 128), jnp.float32),
            pltpu.VMEM((1, 16) if gather_rows else (GN, kc), jnp.float32),
            pltpu.VMEM((1, 16) if gather_rows else (GN, kc), jnp.float32),
            pltpu.VMEM((kc,), jnp.float32),
            pltpu.VMEM((NP,), jnp.float32),
            pltpu.VMEM_SHARED((NP, 128), jnp.float32),
            pltpu.SemaphoreType.DMA,
            pltpu.SemaphoreType.DMA,
            pltpu.SemaphoreType.DMA,
            pltpu.SemaphoreType.DMA,
            pltpu.SemaphoreType.DMA,
            pltpu.SemaphoreType.DMA,
        ],
    )
    return f


def _sck_nodes(scale):
    def body(rows_h, tg_h, nb_h, gid_h, z128_h, z1_h, *rest):
        if scale:
            (outC, outS, outSP, tg_v, idx_v, rows_v, qv, wv, s_v, sp_v,
             sred_v, acc_sh) = rest
        else:
            (outC, idx_v, rows_v, acc_sh) = rest
        c = lax.axis_index("c")
        t = lax.axis_index("s")
        zr = NB // NS
        pltpu.sync_copy(z128_h.at[pl.ds(0, zr)], acc_sh.at[pl.ds(t * zr, zr)])
        if scale:
            pltpu.sync_copy(z1_h.at[pl.ds(0, NB)], s_v)
            pltpu.sync_copy(tg_h.at[c], tg_v)
        plsc.subcore_barrier()

        nbase = t * NPT
        fbase = c * NP + nbase
        pltpu.sync_copy(gid_h.at[c].at[pl.ds(nbase, NPT)], idx_v)
        pltpu.sync_copy(rows_h.at[pl.ds(fbase, NPT)], rows_v)
        if scale:
            pltpu.sync_copy(nb_h.at[pl.ds(fbase, NPT)], qv)

            def wbody(i, _):
                sl = pl.ds(i * 16, 16)
                d16 = idx_v[sl]
                pd = plsc.load_gather(tg_v, [d16])
                lo = pd + qv[sl]
                lo = jnp.where(lo >= 0, lo, 0.01 * lo)
                w16 = jnp.exp(lo)
                wv[sl] = w16
                plsc.addupdate_scatter(s_v, [d16], w16)
                return 0

            lax.fori_loop(0, NPT // 16, wbody, 0)
            _scale_loop(rows_v, wv, NPT, 128)
        pltpu.sync_copy(rows_v, acc_sh.at[idx_v], add=True)
        if scale:
            pltpu.sync_copy(s_v, outSP.at[c].at[t])
        plsc.subcore_barrier()
        pltpu.sync_copy(acc_sh.at[pl.ds(t * zr, zr)],
                        outC.at[c].at[pl.ds(t * zr, zr)])
        if scale:
            @pl.when(t == 0)
            def _reduce():
                pltpu.sync_copy(outSP.at[c], sp_v)

                def redk(i, _):
                    sl = pl.ds(i * 16, 16)
                    a = sp_v[0, sl]
                    for r in range(1, NS):
                        a = a + sp_v[r, sl]
                    sred_v[sl] = a
                    return 0

                lax.fori_loop(0, NB // 16, redk, 0)
                pltpu.sync_copy(sred_v, outS.at[c])

    if scale:
        out_type = (jax.ShapeDtypeStruct((2, NB, 128), jnp.float32),
                    jax.ShapeDtypeStruct((2, NB), jnp.float32),
                    jax.ShapeDtypeStruct((2, NS, NB), jnp.float32))
        scratch = [
            pltpu.VMEM((NB,), jnp.float32),
            pltpu.VMEM((NPT,), jnp.int32),
            pltpu.VMEM((NPT, 128), jnp.float32),
            pltpu.VMEM((NPT,), jnp.float32),
            pltpu.VMEM((NPT,), jnp.float32),
            pltpu.VMEM((NB,), jnp.float32),
            pltpu.VMEM((NS, NB), jnp.float32),
            pltpu.VMEM((NB,), jnp.float32),
            pltpu.VMEM_SHARED((NB, 128), jnp.float32),
        ]
    else:
        out_type = jax.ShapeDtypeStruct((2, NB, 128), jnp.float32)
        scratch = [
            pltpu.VMEM((NPT,), jnp.int32),
            pltpu.VMEM((NPT, 128), jnp.float32),
            pltpu.VMEM_SHARED((NB, 128), jnp.float32),
        ]
    return pl.kernel(body, out_type=out_type, mesh=_get_mesh(),
        compiler_params=pltpu.CompilerParams(needs_layout_passes=False),
                     scratch_types=scratch)


def _tck_pre(x, Wpn, bpn, u, W1a, b1):
    def body(x_r, W_r, b_r, u_r, Wa_r, b1_r, hv_r, p_r, xa_r):
        xb = x_r[...]
        h = _leaky(jnp.dot(xb, W_r[...],
                           preferred_element_type=jnp.float32) + b_r[...])
        hv_r[...] = h
        p_r[...] = jnp.dot(h, u_r[...], preferred_element_type=jnp.float32)
        xa_r[...] = jnp.dot(xb, Wa_r[...],
                            preferred_element_type=jnp.float32) + b1_r[...]

    return pl.pallas_call(
        body,
        grid=(SN // RB,),
        in_specs=[
            pl.BlockSpec((RB, 32), lambda i: (i, 0)),
            pl.BlockSpec((32, 128), lambda i: (0, 0)),
            pl.BlockSpec((1, 128), lambda i: (0, 0)),
            pl.BlockSpec((128, 1), lambda i: (0, 0)),
            pl.BlockSpec((32, 128), lambda i: (0, 0)),
            pl.BlockSpec((1, 128), lambda i: (0, 0)),
        ],
        out_specs=[
            pl.BlockSpec((RB, 128), lambda i: (i, 0)),
            pl.BlockSpec((RB, 1), lambda i: (i, 0)),
            pl.BlockSpec((RB, 128), lambda i: (i, 0)),
        ],
        out_shape=[
            jax.ShapeDtypeStruct((SN, 128), jnp.float32),
            jax.ShapeDtypeStruct((SN, 1), jnp.float32),
            jax.ShapeDtypeStruct((SN, 128), jnp.float32),
        ],
    )(x, Wpn, bpn, u, W1a, b1)


def _tck_edge_mlp(xa_src, eT, Wb, v, b2):
    def body(xa_r, e_r, Wb_r, v_r, b2_r, he_r, q_r):
        eb = lax.dot_general(e_r[...], Wb_r[...],
                             (((0,), (0,)), ((), ())),
                             preferred_element_type=jnp.float32)
        h = _leaky(xa_r[...] + eb)
        he_r[...] = h
        q2 = jnp.dot(h, v_r[...],
                     preferred_element_type=jnp.float32) + b2_r[...]
        q_r[...] = q2.reshape(REB // KC, KC)

    return pl.pallas_call(
        body,
        grid=(SE // REB,),
        in_specs=[
            pl.BlockSpec((REB, 128), lambda i: (i, 0)),
            pl.BlockSpec((6, REB), lambda i: (0, i)),
            pl.BlockSpec((6, 128), lambda i: (0, 0)),
            pl.BlockSpec((128, 1), lambda i: (0, 0)),
            pl.BlockSpec((1, 1), lambda i: (0, 0)),
        ],
        out_specs=[
            pl.BlockSpec((REB, 128), lambda i: (i, 0)),
            pl.BlockSpec((REB // KC, KC), lambda i: (i, 0)),
        ],
        out_shape=[
            jax.ShapeDtypeStruct((SE, 128), jnp.float32),
            jax.ShapeDtypeStruct((SE // KC, KC), jnp.float32),
        ],
    )(xa_src, eT, Wb, v, b2)


def _tck_post_gc(Cw, sw, hvnew, Wet, bet, WihT, WhhT, bih, bhh,
                 u1, u2, bu2, Wpnode, bpnode):
    def body(Cw_r, s_r, hv_r, Wet_r, bet_r, WihT_r, WhhT_r, bih_r, bhh_r,
             u1_r, u2_r, bu2_r, Wpn_r, bpn_r,
             node_r, hvo_r, p1_r, p2_r):
        s = s_r[...]
        inv = 1.0 / (s + EPS)
        sn = s * inv
        c = jnp.dot(Cw_r[...], Wet_r[...],
                    preferred_element_type=jnp.float32) * inv + bet_r[...] * sn
        node = jax.nn.relu(_gru_tc(_elu(c), hv_r[...], WihT_r[...],
                                   WhhT_r[...], bih_r[...], bhh_r[...]))
        node_r[...] = node
        hvo_r[...] = jnp.dot(node, Wpn_r[...],
                             preferred_element_type=jnp.float32) + bpn_r[...]
        p1_r[...] = jnp.dot(node, u1_r[...], preferred_element_type=jnp.float32)
        p2_r[...] = jnp.dot(node, u2_r[...],
                            preferred_element_type=jnp.float32) + bu2_r[...]

    return pl.pallas_call(
        body,
        grid=(SN // RB,),
        in_specs=[
            pl.BlockSpec((RB, 128), lambda i: (i, 0)),
            pl.BlockSpec((RB, 1), lambda i: (i, 0)),
            pl.BlockSpec((RB, 128), lambda i: (i, 0)),
            pl.BlockSpec((128, 128), lambda i: (0, 0)),
            pl.BlockSpec((1, 128), lambda i: (0, 0)),
            pl.BlockSpec((128, 384), lambda i: (0, 0)),
            pl.BlockSpec((128, 384), lambda i: (0, 0)),
            pl.BlockSpec((1, 384), lambda i: (0, 0)),
            pl.BlockSpec((1, 384), lambda i: (0, 0)),
            pl.BlockSpec((128, 1), lambda i: (0, 0)),
            pl.BlockSpec((128, 1), lambda i: (0, 0)),
            pl.BlockSpec((1, 1), lambda i: (0, 0)),
            pl.BlockSpec((128, 128), lambda i: (0, 0)),
            pl.BlockSpec((1, 128), lambda i: (0, 0)),
        ],
        out_specs=[
            pl.BlockSpec((RB, 128), lambda i: (i, 0)),
            pl.BlockSpec((RB, 128), lambda i: (i, 0)),
            pl.BlockSpec((RB, 1), lambda i: (i, 0)),
            pl.BlockSpec((RB, 1), lambda i: (i, 0)),
        ],
        out_shape=[
            jax.ShapeDtypeStruct((SN, 128), jnp.float32),
            jax.ShapeDtypeStruct((SN, 128), jnp.float32),
            jax.ShapeDtypeStruct((SN, 1), jnp.float32),
            jax.ShapeDtypeStruct((SN, 1), jnp.float32),
        ],
    )(Cw, sw, hvnew, Wet, bet, WihT, WhhT, bih, bhh, u1, u2, bu2,
      Wpnode, bpnode)


def _tck_post_l1(C2, s2, node, WihT, WhhT, bih, bhh,
                 Wpn0, bpn0, Wpn1, bpn1, wb0, bcl0, wb1, bcl1):
    def body(C_r, s_r, nd_r, WihT_r, WhhT_r, bih_r, bhh_r,
             Wpn0_r, bpn0_r, Wpn1_r, bpn1_r, wb0_r, bcl0_r, wb1_r, bcl1_r,
             n2_r, hv0_r, hv1_r, nb0_r, nb1_r):
        c = C_r[...] / (s_r[...] + EPS)
        n2 = jax.nn.relu(_gru_tc(_elu(c), nd_r[...], WihT_r[...],
                                 WhhT_r[...], bih_r[...], bhh_r[...]))
        n2_r[...] = n2
        hv0_r[...] = jnp.dot(n2, Wpn0_r[...],
                             preferred_element_type=jnp.float32) + bpn0_r[...]
        hv1_r[...] = jnp.dot(n2, Wpn1_r[...],
                             preferred_element_type=jnp.float32) + bpn1_r[...]
        nb0_r[...] = jnp.dot(n2, wb0_r[...],
                             preferred_element_type=jnp.float32) + bcl0_r[...]
        nb1_r[...] = jnp.dot(n2, wb1_r[...],
                             preferred_element_type=jnp.float32) + bcl1_r[...]

    return pl.pallas_call(
        body,
        grid=(SN // RB,),
        in_specs=[
            pl.BlockSpec((RB, 128), lambda i: (i, 0)),
            pl.BlockSpec((RB, 1), lambda i: (i, 0)),
            pl.BlockSpec((RB, 128), lambda i: (i, 0)),
            pl.BlockSpec((128, 384), lambda i: (0, 0)),
            pl.BlockSpec((128, 384), lambda i: (0, 0)),
            pl.BlockSpec((1, 384), lambda i: (0, 0)),
            pl.BlockSpec((1, 384), lambda i: (0, 0)),
            pl.BlockSpec((128, 128), lambda i: (0, 0)),
            pl.BlockSpec((1, 128), lambda i: (0, 0)),
            pl.BlockSpec((128, 128), lambda i: (0, 0)),
            pl.BlockSpec((1, 128), lambda i: (0, 0)),
            pl.BlockSpec((128, 1), lambda i: (0, 0)),
            pl.BlockSpec((1, 1), lambda i: (0, 0)),
            pl.BlockSpec((128, 1), lambda i: (0, 0)),
            pl.BlockSpec((1, 1), lambda i: (0, 0)),
        ],
        out_specs=[pl.BlockSpec((RB, 128), lambda i: (i, 0))] * 3
        + [pl.BlockSpec((RB, 1), lambda i: (i, 0))] * 2,
        out_shape=[jax.ShapeDtypeStruct((SN, 128), jnp.float32)] * 3
        + [jax.ShapeDtypeStruct((SN, 1), jnp.float32)] * 2,
    )(C2, s2, node, WihT, WhhT, bih, bhh, Wpn0, bpn0, Wpn1, bpn1,
      wb0, bcl0, wb1, bcl1)


def _tck_tg(G, wa):
    def body(G_r, wa_r, tg_r):
        tg_r[...] = jnp.dot(jax.nn.relu(G_r[...]), wa_r[...],
                            preferred_element_type=jnp.float32)

    return pl.pallas_call(
        body,
        grid=(1,),
        in_specs=[
            pl.BlockSpec((2 * NB, 128), lambda i: (0, 0)),
            pl.BlockSpec((128, 1), lambda i: (0, 0)),
        ],
        out_specs=pl.BlockSpec((2 * NB, 1), lambda i: (0, 0)),
        out_shape=jax.ShapeDtypeStruct((2 * NB, 1), jnp.float32),
    )(G, wa)


def _tck_ro_gru(G, s, h, WihT, WhhT, bih, bhh, wa_next):
    def body(G_r, s_r, h_r, WihT_r, WhhT_r, bih_r, bhh_r, wa_r, g_r, tg_r):
        g_repr = _elu(G_r[...] / (s_r[...] + EPS))
        g = jax.nn.relu(_gru_tc(g_repr, h_r[...], WihT_r[...], WhhT_r[...],
                                bih_r[...], bhh_r[...]))
        g_r[...] = g
        tg_r[...] = jnp.dot(jax.nn.relu(g), wa_r[...],
                            preferred_element_type=jnp.float32)

    return pl.pallas_call(
        body,
        grid=(1,),
        in_specs=[
            pl.BlockSpec((2 * NB, 128), lambda i: (0, 0)),
            pl.BlockSpec((2 * NB, 1), lambda i: (0, 0)),
            pl.BlockSpec((2 * NB, 128), lambda i: (0, 0)),
            pl.BlockSpec((128, 384), lambda i: (0, 0)),
            pl.BlockSpec((128, 384), lambda i: (0, 0)),
            pl.BlockSpec((1, 384), lambda i: (0, 0)),
            pl.BlockSpec((1, 384), lambda i: (0, 0)),
            pl.BlockSpec((128, 1), lambda i: (0, 0)),
        ],
        out_specs=[
            pl.BlockSpec((2 * NB, 128), lambda i: (0, 0)),
            pl.BlockSpec((2 * NB, 1), lambda i: (0, 0)),
        ],
        out_shape=[
            jax.ShapeDtypeStruct((2 * NB, 128), jnp.float32),
            jax.ShapeDtypeStruct((2 * NB, 1), jnp.float32),
        ],
    )(G, s, h, WihT, WhhT, bih, bhh, wa_next)


def _tck_final(G, s, h, WihT, WhhT, bih, bhh, Wpred, bpred,
               WfcA, WfcB, bfc, bn_a, bn_b, Wout, bout):
    def body(G_r, s_r, h_r, WihT_r, WhhT_r, bih_r, bhh_r, Wp_r, bp_r,
             WA_r, WB_r, bfc_r, bna_r, bnb_r, Wo_r, bo_r, o_r):
        g_repr = _elu(G_r[...] / (s_r[...] + EPS))
        g = jax.nn.relu(_gru_tc(g_repr, h_r[...], WihT_r[...], WhhT_r[...],
                                bih_r[...], bhh_r[...]))
        pred = jnp.dot(g, Wp_r[...],
                       preferred_element_type=jnp.float32) + bp_r[...]
        s1 = pred[0:B, :]
        s2 = pred[NB:NB + B, :]
        hh = (jnp.dot(s1, WA_r[...], preferred_element_type=jnp.float32)
              + jnp.dot(s2, WB_r[...], preferred_element_type=jnp.float32)
              + bfc_r[...])
        hh = jax.nn.relu(hh * bna_r[...] + bnb_r[...])
        o_r[...] = jnp.dot(hh, Wo_r[...],
                           preferred_element_type=jnp.float32) + bo_r[...]

    return pl.pallas_call(
        body,
        grid=(1,),
        in_specs=[
            pl.BlockSpec((2 * NB, 128), lambda i: (0, 0)),
            pl.BlockSpec((2 * NB, 1), lambda i: (0, 0)),
            pl.BlockSpec((2 * NB, 128), lambda i: (0, 0)),
            pl.BlockSpec((128, 384), lambda i: (0, 0)),
            pl.BlockSpec((128, 384), lambda i: (0, 0)),
            pl.BlockSpec((1, 384), lambda i: (0, 0)),
            pl.BlockSpec((1, 384), lambda i: (0, 0)),
            pl.BlockSpec((128, 256), lambda i: (0, 0)),
            pl.BlockSpec((1, 256), lambda i: (0, 0)),
            pl.BlockSpec((256, 1024), lambda i: (0, 0)),
            pl.BlockSpec((256, 1024), lambda i: (0, 0)),
            pl.BlockSpec((1, 1024), lambda i: (0, 0)),
            pl.BlockSpec((1, 1024), lambda i: (0, 0)),
            pl.BlockSpec((1, 1024), lambda i: (0, 0)),
            pl.BlockSpec((1024, 1), lambda i: (0, 0)),
            pl.BlockSpec((1, 1), lambda i: (0, 0)),
        ],
        out_specs=pl.BlockSpec((B, 1), lambda i: (0, 0)),
        out_shape=jax.ShapeDtypeStruct((B, 1), jnp.float32),
    )(G, s, h, WihT, WhhT, bih, bhh, Wpred, bpred, WfcA, WfcB, bfc,
      bn_a, bn_b, Wout, bout)


def kernel(x1, e1, edge_index1, gid1, x2, e2, edge_index2, gid2, Wfc, Wout,
           Wpred, bfc, bn_beta, bn_gamma, bn_mean, bn_var, bout, bpred,
           gc_Wet, gc_Whh, gc_Wih, gc_Wpe1, gc_Wpe2, gc_Wpn, gc_bet, gc_bhh,
           gc_bih, gc_bpe1, gc_bpe2, gc_bpn, l1_Whh, l1_Wih, l1_Wpe,
           l1_Wpnode, l1_bhh, l1_bih, l1_bpe, l1_bpnode, ro0_Wcl, ro0_Whh,
           ro0_Wih, ro0_Wpn, ro0_bcl, ro0_bhh, ro0_bih, ro0_bpn, ro1_Wcl,
           ro1_Whh, ro1_Wih, ro1_Wpn, ro1_bcl, ro1_bhh, ro1_bih, ro1_bpn):
    f32 = jnp.float32
    pad_n = NP - N
    pad_e = EP - E
    xs_pad = lambda a: jnp.pad(a, ((0, pad_n), (0, 0)))
    ep2 = lambda a: jnp.pad(a, ((0, pad_e), (0, 0)))
    ep1 = lambda a, v=0: jnp.pad(a, (0, pad_e), constant_values=v)
    x2n = jnp.concatenate([xs_pad(x1), xs_pad(x2)], axis=0)
    eT = jnp.concatenate([ep2(e1), ep2(e2)], axis=0).T
    src_b = jnp.concatenate([ep1(edge_index1[0]),
                             ep1(edge_index2[0]) + NP])
    dst2d = jnp.stack([ep1(edge_index1[1], N),
                       ep1(edge_index2[1], N)])
    gid_pad = lambda g: jnp.pad(g, (0, pad_n), constant_values=B)
    gid2d = jnp.stack([gid_pad(gid1), gid_pad(gid2)])
    z128 = jnp.zeros((NP, 128), f32)
    z1 = jnp.zeros((NP,), f32)

    row = lambda b: b.reshape(1, -1)
    col = lambda w: w.reshape(-1, 1)
    u_gc = col(gc_Wpe2[:128, 0])
    v_gc = col(gc_Wpe2[128:, 0])
    u1_l1 = col(l1_Wpe[:128, 0])
    u2_l1 = col(l1_Wpe[128:, 0])
    wa0, wb0 = col(ro0_Wcl[:128, 0]), col(ro0_Wcl[128:, 0])
    wa1, wb1 = col(ro1_Wcl[:128, 0]), col(ro1_Wcl[128:, 0])
    bn_a = row(bn_gamma / jnp.sqrt(bn_var + 1e-5))
    bn_b = row(bn_beta - bn_mean * bn_gamma / jnp.sqrt(bn_var + 1e-5))
    WfcA, WfcB = Wfc[:256], Wfc[256:]

    hv_new, p_gc, xa = _tck_pre(x2n, gc_Wpn, row(gc_bpn), u_gc,
                                gc_Wpe1[:32], row(gc_bpe1))
    xa_src = _sck_gather_rows(xa.reshape(2, NP, 128), src_b, 128)
    he1, q_gc = _tck_edge_mlp(xa_src, eT, gc_Wpe1[32:],
                              v_gc, gc_bpe2.reshape(1, 1))
    p2d_gc = p_gc.reshape(2, NP)
    edge_gc = _sck_edge(gather_rows=False)
    dst3 = dst2d.reshape(2, EP // KC, KC)
    src3 = src_b.reshape(SE // KC, KC)
    Cw, sw, _sp1 = edge_gc(he1, q_gc, p2d_gc, dst3,
                           src3, z128, z1)
    node, hv_l1, p1, p2 = _tck_post_gc(
        Cw.reshape(SN, 128), sw.reshape(SN, 1), hv_new,
        gc_Wet, row(gc_bet), gc_Wih.T, gc_Whh.T, row(gc_bih), row(gc_bhh),
        u1_l1, u2_l1, l1_bpe.reshape(1, 1), l1_Wpnode, row(l1_bpnode))

    edge_l1 = _sck_edge(gather_rows=True)
    dst3b = dst2d.reshape(2, EP // 32, 32)
    src3b = src_b.reshape(SE // 32, 32)
    C2, s2, _sp2 = edge_l1(hv_l1, p2.reshape(2, NP), p1.reshape(2, NP),
                           dst3b, src3b, z128, z1)
    node2, hv0, hv1, nb0, nb1 = _tck_post_l1(
        C2.reshape(SN, 128), s2.reshape(SN, 1), node,
        l1_Wih.T, l1_Whh.T, row(l1_bih), row(l1_bhh),
        ro0_Wpn, row(ro0_bpn), ro1_Wpn, row(ro1_bpn),
        wb0, ro0_bcl.reshape(1, 1), wb1, ro1_bcl.reshape(1, 1))

    zt = jnp.zeros((2, NB), f32)
    zn = jnp.zeros((SN,), f32)
    gf = _sck_nodes(scale=False)(node2, zt, zn, gid2d, z128, z1)
    gfeats = gf.reshape(2 * NB, 128)
    tg0 = _tck_tg(gfeats, wa0)
    ro_k = _sck_nodes(scale=True)
    G0, S0, _sp3 = ro_k(hv0, tg0.reshape(2, NB), nb0.reshape(SN), gid2d, z128, z1)
    gf1, tg1 = _tck_ro_gru(G0.reshape(2 * NB, 128),
                           S0.reshape(2 * NB, 1), gfeats,
                           ro0_Wih.T, ro0_Whh.T, row(ro0_bih), row(ro0_bhh),
                           wa1)
    G1, S1, _sp4 = ro_k(hv1, tg1.reshape(2, NB), nb1.reshape(SN), gid2d, z128, z1)
    o = _tck_final(G1.reshape(2 * NB, 128), S1.reshape(2 * NB, 1),
                   gf1, ro1_Wih.T, ro1_Whh.T, row(ro1_bih), row(ro1_bhh),
                   Wpred, row(bpred), WfcA, WfcB, row(bfc), bn_a, bn_b,
                   Wout, bout.reshape(1, 1))
    return o.reshape(B)


# speedup vs baseline: 1.2615x; 1.0031x over previous
import functools

import jax
import jax.numpy as jnp
from jax import lax
from jax.experimental import pallas as pl
from jax.experimental.pallas import tpu as pltpu
from jax.experimental.pallas import tpu_sc as plsc

N, E, B = 10000, 160000, 512
NP = 10240
SN = 2 * NP
EP = 163840
SE = 2 * EP
NC, NS = 2, 16
EPT = EP // NS
KC = 64
NPT = NP // NS
NB = 640
EPS = 1e-9
RB = 2048
REB = 2048

@functools.cache
def _get_mesh():
    return plsc.VectorSubcoreMesh(core_axis_name="c", subcore_axis_name="s",
                                  num_cores=NC, num_subcores=NS)


def _leaky(x):
    return jnp.where(x >= 0, x, 0.01 * x)


def _elu(x):
    return jnp.where(x > 0, x, jnp.exp(x) - 1.0)


def _gru_tc(x, h, wihT, whhT, bih, bhh):
    gi = jnp.dot(x, wihT, preferred_element_type=jnp.float32) + bih
    gh = jnp.dot(h, whhT, preferred_element_type=jnp.float32) + bhh
    r = jax.nn.sigmoid(gi[:, 0:128] + gh[:, 0:128])
    z = jax.nn.sigmoid(gi[:, 128:256] + gh[:, 128:256])
    n = jnp.tanh(gi[:, 256:384] + r * gh[:, 256:384])
    return (1.0 - z) * n + z * h


def _sck_gather_rows(table3, idx, d, dtype=jnp.float32):
    kc = 128
    nch = EPT // kc

    def body(tab_h, idx_h, out_h, idx0, idx1, rows0, rows1, tab_sh,
             g0, g1, s0, s1):
        c = lax.axis_index("c")
        t = lax.axis_index("s")
        idx_b = [idx0, idx1]
        rows_b = [rows0, rows1]
        gsem = [g0, g1]
        ssem = [s0, s1]
        pltpu.sync_copy(tab_h.at[c].at[pl.ds(t * NPT, NPT)],
                        tab_sh.at[pl.ds(t * NPT, NPT)])
        plsc.subcore_barrier()
        base = c * EP + t * EPT

        def start_gather(k, b):
            pltpu.sync_copy(idx_h.at[pl.ds(base + k * kc, kc)], idx_b[b])

            def adj(i, _):
                sl = pl.ds(i * 16, 16)
                idx_b[b][sl] = idx_b[b][sl] - c * NP
                return 0

            lax.fori_loop(0, kc // 16, adj, 0)
            pltpu.async_copy(tab_sh.at[idx_b[b]], rows_b[b], gsem[b])

        start_gather(0, 0)

        def pair(g, _):
            for b in range(2):
                k = g * 2 + b
                nb = 1 - b
                pltpu.make_async_copy(tab_sh.at[idx_b[b]], rows_b[b],
                                      gsem[b]).wait()

                @pl.when(k + 1 < nch)
                def _pre():
                    @pl.when(k >= 1)
                    def _drain():
                        pltpu.make_async_copy(
                            rows_b[nb], out_h.at[pl.ds(base, kc)],
                            ssem[nb]).wait()
                    start_gather(k + 1, nb)

                pltpu.async_copy(rows_b[b], out_h.at[pl.ds(base + k * kc, kc)],
                                 ssem[b])
            return 0

        lax.fori_loop(0, nch // 2, pair, 0)
        pltpu.make_async_copy(rows_b[0], out_h.at[pl.ds(base, kc)],
                              ssem[0]).wait()
        pltpu.make_async_copy(rows_b[1], out_h.at[pl.ds(base, kc)],
                              ssem[1]).wait()

    f = pl.kernel(
        body,
        out_type=jax.ShapeDtypeStruct((SE, d), dtype),
        mesh=_get_mesh(),
        compiler_params=pltpu.CompilerParams(needs_layout_passes=False),
        scratch_types=[
            pltpu.VMEM((kc,), jnp.int32),
            pltpu.VMEM((kc,), jnp.int32),
            pltpu.VMEM((kc, d), dtype),
            pltpu.VMEM((kc, d), dtype),
            pltpu.VMEM_SHARED((NP, d), dtype),
            pltpu.SemaphoreType.DMA,
            pltpu.SemaphoreType.DMA,
            pltpu.SemaphoreType.DMA,
            pltpu.SemaphoreType.DMA,
        ],
    )
    return f(table3, idx)


def _scale_loop(rows_v, wv, kc, dcols):
    def sbody(j, _):
        wj = plsc.load_gather(wv, [jnp.full((16,), j, jnp.int32)])
        for f in range(dcols // 16):
            sl = pl.ds(f * 16, 16)
            rows_v[j, sl] = rows_v[j, sl] * wj
        return 0

    lax.fori_loop(0, kc, sbody, 0)


GN = 4
PV = 10112


def _sck_edge(gather_rows):
    kc = 32 if gather_rows else KC
    nch = EPT // kc
    ngrp = nch // GN
    CPT = EPT // kc

    def body(rows_h, q_h, p_h, dst_h, src_h, z128_h, z1_h, outC, outS, outSP,
             p_v, ps_v, idxd0, idxd1, idxs0, idxs1, rows0, rows1, qg0, qg1,
             wv, s_v, acc_sh, i0, i1, g0, g1, c0, c1):
        c = lax.axis_index("c")
        t = lax.axis_index("s")
        idxd_g = [idxd0, idxd1]
        idxs_g = [idxs0, idxs1]
        rows_b = [rows0, rows1]
        qg_b = [qg0, qg1]
        isem = [i0, i1]
        gsem = [g0, g1]
        csem = [c0, c1]
        pltpu.sync_copy(z128_h.at[pl.ds(t * NPT, NPT)],
                        acc_sh.at[pl.ds(t * NPT, NPT)])
        pltpu.sync_copy(z1_h, s_v)
        pltpu.sync_copy(p_h.at[c].at[pl.ds(0, PV)], p_v)
        if gather_rows:
            pltpu.sync_copy(q_h.at[c].at[pl.ds(0, PV)], ps_v)
        plsc.subcore_barrier()

        def g_idx_load(m, buf):
            crow = t * CPT + m * GN
            frow = c * CPT * NS + crow
            pltpu.async_copy(dst_h.at[c].at[pl.ds(crow, GN)], idxd_g[buf],
                             isem[buf])
            if gather_rows:
                pltpu.async_copy(src_h.at[pl.ds(frow, GN)], idxs_g[buf],
                                 isem[buf])
            else:
                pltpu.async_copy(q_h.at[pl.ds(frow, GN)], qg_b[buf],
                                 isem[buf])

        def wait_gidx(buf):
            pltpu.make_async_copy(dst_h.at[c].at[pl.ds(0, GN)], idxd_g[buf],
                                  isem[buf]).wait()
            if gather_rows:
                pltpu.make_async_copy(src_h.at[pl.ds(0, GN)], idxs_g[buf],
                                      isem[buf]).wait()
            else:
                pltpu.make_async_copy(q_h.at[pl.ds(0, GN)], qg_b[buf],
                                      isem[buf]).wait()

        def start_rows(k, gb, j, rb):
            if gather_rows:
                pltpu.async_copy(rows_h.at[idxs_g[gb].at[j]], rows_b[rb],
                                 gsem[rb])
            else:
                fbase = c * EP + t * EPT + k * kc
                pltpu.async_copy(rows_h.at[pl.ds(fbase, KC)], rows_b[rb],
                                 gsem[rb])

        def wait_rows(rb):
            if gather_rows:
                pltpu.make_async_copy(rows_h.at[idxs_g[0].at[0]], rows_b[rb],
                                      gsem[rb]).wait()
            else:
                pltpu.make_async_copy(rows_h.at[pl.ds(0, kc)], rows_b[rb],
                                      gsem[rb]).wait()

        def wait_scat(rb):
            pltpu.make_async_copy(rows_b[rb], acc_sh.at[idxd_g[0].at[0]],
                                  csem[rb]).wait()

        g_idx_load(0, 0)
        wait_gidx(0)
        start_rows(0, 0, 0, 0)
        g_idx_load(1, 1)

        def gpair(gp, _):
            for gpar in range(2):
                g = gp * 2 + gpar
                gb = gpar
                for j in range(GN):
                    k = g * GN + j
                    rb = j % 2
                    nrb = 1 - rb
                    wait_rows(rb)
                    if j == 1:
                        @pl.when(jnp.logical_and(g >= 1, g + 1 < ngrp))
                        def _ld():
                            g_idx_load(g + 1, 1 - gb)

                    @pl.when(k + 1 < nch)
                    def _pre():
                        @pl.when(k >= 1)
                        def _drain():
                            wait_scat(nrb)
                        if j == GN - 1:
                            wait_gidx(1 - gb)
                            start_rows(k + 1, 1 - gb, 0, nrb)
                        else:
                            start_rows(k + 1, gb, j + 1, nrb)

                    def wbody(i, _):
                        sl = pl.ds(i * 16, 16)
                        d16 = idxd_g[gb][j, sl]
                        pd = plsc.load_gather(p_v, [d16])
                        if gather_rows:
                            qq = plsc.load_gather(
                                ps_v, [idxs_g[gb][j, sl] - c * NP])
                        else:
                            qq = qg_b[gb][j, sl]
                        lo = pd + qq
                        lo = jnp.where(lo >= 0, lo, 0.01 * lo)
                        w16 = jnp.exp(lo)
                        wv[sl] = w16
                        plsc.addupdate_scatter(s_v, [d16], w16)
                        return 0

                    lax.fori_loop(0, kc // 16, wbody, 0)
                    _scale_loop(rows_b[rb], wv, kc, 128)
                    pltpu.async_copy(rows_b[rb], acc_sh.at[idxd_g[gb].at[j]],
                                     csem[rb], add=True)
            return 0

        lax.fori_loop(0, ngrp // 2, gpair, 0)
        wait_scat(0)
        wait_scat(1)
        pltpu.sync_copy(s_v, outSP.at[c].at[t])
        plsc.subcore_barrier()
        pltpu.sync_copy(acc_sh.at[pl.ds(t * NPT, NPT)],
                        outC.at[c].at[pl.ds(t * NPT, NPT)])

        def redk(kk, _):
            off = t * NPT + kk * 128
            pltpu.sync_copy(outSP.at[c].at[:, pl.ds(off, 128)],
                            rows0.at[pl.ds(0, NS)])
            for ff in range(8):
                sl = pl.ds(ff * 16, 16)
                a = rows0[0, sl]
                for r in range(1, NS):
                    a = a + rows0[r, sl]
                rows0[NS, sl] = a
            pltpu.sync_copy(rows0.at[NS], outS.at[c].at[pl.ds(off, 128)])
            return 0

        lax.fori_loop(0, NPT // 128, redk, 0)

    f = pl.kernel(
        body,
        out_type=(jax.ShapeDtypeStruct((2, NP, 128), jnp.float32),
                  jax.ShapeDtypeStruct((2, NP), jnp.float32),
                  jax.ShapeDtypeStruct((2, NS, NP), jnp.float32)),
        mesh=_get_mesh(),
        compiler_params=pltpu.CompilerParams(needs_layout_passes=False),
        scratch_types=[
            pltpu.VMEM((PV,), jnp.float32),
            pltpu.VMEM((PV,) if gather_rows else (16,), jnp.float32),
            pltpu.VMEM((GN, kc), jnp.int32),
            pltpu.VMEM((GN, kc), jnp.int32),
            pltpu.VMEM((GN, kc) if gather_rows else (1, 16), jnp.int32),
            pltpu.VMEM((GN, kc) if gather_rows else (1, 16), jnp.int32),
            pltpu.VMEM((kc, 128), jnp.float32),
            pltpu.VMEM((kc, 128), jnp.float32),
            pltpu.VMEM((1, 16) if gather_rows else (GN, kc), jnp.float32),
            pltpu.VMEM((1, 16) if gather_rows else (GN, kc), jnp.float32),
            pltpu.VMEM((kc,), jnp.float32),
            pltpu.VMEM((NP,), jnp.float32),
            pltpu.VMEM_SHARED((NP, 128), jnp.float32),
            pltpu.SemaphoreType.DMA,
            pltpu.SemaphoreType.DMA,
            pltpu.SemaphoreType.DMA,
            pltpu.SemaphoreType.DMA,
            pltpu.SemaphoreType.DMA,
            pltpu.SemaphoreType.DMA,
        ],
    )
    return f


def _sck_nodes(scale):
    def body(rows_h, tg_h, nb_h, gid_h, z128_h, z1_h, *rest):
        if scale:
            (outC, outS, outSP, tg_v, idx_v, rows_v, qv, wv, s_v, sp_v,
             sred_v, acc_sh) = rest
        else:
            (outC, idx_v, rows_v, acc_sh) = rest
        c = lax.axis_index("c")
        t = lax.axis_index("s")
        zr = NB // NS
        pltpu.sync_copy(z128_h.at[pl.ds(0, zr)], acc_sh.at[pl.ds(t * zr, zr)])
        if scale:
            pltpu.sync_copy(z1_h.at[pl.ds(0, NB)], s_v)
            pltpu.sync_copy(tg_h.at[c], tg_v)
        plsc.subcore_barrier()

        nbase = t * NPT
        fbase = c * NP + nbase
        pltpu.sync_copy(gid_h.at[c].at[pl.ds(nbase, NPT)], idx_v)
        pltpu.sync_copy(rows_h.at[pl.ds(fbase, NPT)], rows_v)
        if scale:
            pltpu.sync_copy(nb_h.at[pl.ds(fbase, NPT)], qv)

            def wbody(i, _):
                sl = pl.ds(i * 16, 16)
                d16 = idx_v[sl]
                pd = plsc.load_gather(tg_v, [d16])
                lo = pd + qv[sl]
                lo = jnp.where(lo >= 0, lo, 0.01 * lo)
                w16 = jnp.exp(lo)
                wv[sl] = w16
                plsc.addupdate_scatter(s_v, [d16], w16)
                return 0

            lax.fori_loop(0, NPT // 16, wbody, 0)
            _scale_loop(rows_v, wv, NPT, 128)
        pltpu.sync_copy(rows_v, acc_sh.at[idx_v], add=True)
        if scale:
            pltpu.sync_copy(s_v, outSP.at[c].at[t])
        plsc.subcore_barrier()
        pltpu.sync_copy(acc_sh.at[pl.ds(t * zr, zr)],
                        outC.at[c].at[pl.ds(t * zr, zr)])
        if scale:
            @pl.when(t == 0)
            def _reduce():
                pltpu.sync_copy(outSP.at[c], sp_v)

                def redk(i, _):
                    sl = pl.ds(i * 16, 16)
                    a = sp_v[0, sl]
                    for r in range(1, NS):
                        a = a + sp_v[r, sl]
                    sred_v[sl] = a
                    return 0

                lax.fori_loop(0, NB // 16, redk, 0)
                pltpu.sync_copy(sred_v, outS.at[c])

    if scale:
        out_type = (jax.ShapeDtypeStruct((2, NB, 128), jnp.float32),
                    jax.ShapeDtypeStruct((2, NB), jnp.float32),
                    jax.ShapeDtypeStruct((2, NS, NB), jnp.float32))
        scratch = [
            pltpu.VMEM((NB,), jnp.float32),
            pltpu.VMEM((NPT,), jnp.int32),
            pltpu.VMEM((NPT, 128), jnp.float32),
            pltpu.VMEM((NPT,), jnp.float32),
            pltpu.VMEM((NPT,), jnp.float32),
            pltpu.VMEM((NB,), jnp.float32),
            pltpu.VMEM((NS, NB), jnp.float32),
            pltpu.VMEM((NB,), jnp.float32),
            pltpu.VMEM_SHARED((NB, 128), jnp.float32),
        ]
    else:
        out_type = jax.ShapeDtypeStruct((2, NB, 128), jnp.float32)
        scratch = [
            pltpu.VMEM((NPT,), jnp.int32),
            pltpu.VMEM((NPT, 128), jnp.float32),
            pltpu.VMEM_SHARED((NB, 128), jnp.float32),
        ]
    return pl.kernel(body, out_type=out_type, mesh=_get_mesh(),
        compiler_params=pltpu.CompilerParams(needs_layout_passes=False),
                     scratch_types=scratch)


def _tck_pre(x, Wpn, bpn, u, W1a, b1):
    def body(x_r, W_r, b_r, u_r, Wa_r, b1_r, hv_r, p_r, xa_r):
        xb = x_r[...]
        h = _leaky(jnp.dot(xb, W_r[...],
                           preferred_element_type=jnp.float32) + b_r[...])
        hv_r[...] = h
        p_r[...] = jnp.dot(h, u_r[...], preferred_element_type=jnp.float32)
        xa_r[...] = jnp.dot(xb, Wa_r[...],
                            preferred_element_type=jnp.float32) + b1_r[...]

    return pl.pallas_call(
        body,
        grid=(SN // RB,),
        in_specs=[
            pl.BlockSpec((RB, 32), lambda i: (i, 0)),
            pl.BlockSpec((32, 128), lambda i: (0, 0)),
            pl.BlockSpec((1, 128), lambda i: (0, 0)),
            pl.BlockSpec((128, 1), lambda i: (0, 0)),
            pl.BlockSpec((32, 128), lambda i: (0, 0)),
            pl.BlockSpec((1, 128), lambda i: (0, 0)),
        ],
        out_specs=[
            pl.BlockSpec((RB, 128), lambda i: (i, 0)),
            pl.BlockSpec((RB, 1), lambda i: (i, 0)),
            pl.BlockSpec((RB, 128), lambda i: (i, 0)),
        ],
        out_shape=[
            jax.ShapeDtypeStruct((SN, 128), jnp.float32),
            jax.ShapeDtypeStruct((SN, 1), jnp.float32),
            jax.ShapeDtypeStruct((SN, 128), jnp.float32),
        ],
    )(x, Wpn, bpn, u, W1a, b1)


def _tck_edge_mlp(xa_src, eT, Wb, v, b2):
    def body(xa_r, e_r, Wb_r, v_r, b2_r, he_r, q_r):
        eb = lax.dot_general(e_r[...], Wb_r[...],
                             (((0,), (0,)), ((), ())),
                             preferred_element_type=jnp.float32)
        h = _leaky(xa_r[...] + eb)
        he_r[...] = h
        q2 = jnp.dot(h, v_r[...],
                     preferred_element_type=jnp.float32) + b2_r[...]
        q_r[...] = q2.reshape(REB // KC, KC)

    return pl.pallas_call(
        body,
        grid=(SE // REB,),
        in_specs=[
            pl.BlockSpec((REB, 128), lambda i: (i, 0)),
            pl.BlockSpec((6, REB), lambda i: (0, i)),
            pl.BlockSpec((6, 128), lambda i: (0, 0)),
            pl.BlockSpec((128, 1), lambda i: (0, 0)),
            pl.BlockSpec((1, 1), lambda i: (0, 0)),
        ],
        out_specs=[
            pl.BlockSpec((REB, 128), lambda i: (i, 0)),
            pl.BlockSpec((REB // KC, KC), lambda i: (i, 0)),
        ],
        out_shape=[
            jax.ShapeDtypeStruct((SE, 128), jnp.float32),
            jax.ShapeDtypeStruct((SE // KC, KC), jnp.float32),
        ],
    )(xa_src, eT, Wb, v, b2)


def _tck_post_gc(Cw, sw, hvnew, Wet, bet, WihT, WhhT, bih, bhh,
                 u1, u2, bu2, Wpnode, bpnode):
    def body(Cw_r, s_r, hv_r, Wet_r, bet_r, WihT_r, WhhT_r, bih_r, bhh_r,
             u1_r, u2_r, bu2_r, Wpn_r, bpn_r,
             node_r, hvo_r, p1_r, p2_r):
        s = s_r[...]
        inv = 1.0 / (s + EPS)
        sn = s * inv
        c = jnp.dot(Cw_r[...], Wet_r[...],
                    preferred_element_type=jnp.float32) * inv + bet_r[...] * sn
        node = jax.nn.relu(_gru_tc(_elu(c), hv_r[...], WihT_r[...],
                                   WhhT_r[...], bih_r[...], bhh_r[...]))
        node_r[...] = node
        hvo_r[...] = jnp.dot(node, Wpn_r[...],
                             preferred_element_type=jnp.float32) + bpn_r[...]
        p1_r[...] = jnp.dot(node, u1_r[...], preferred_element_type=jnp.float32)
        p2_r[...] = jnp.dot(node, u2_r[...],
                            preferred_element_type=jnp.float32) + bu2_r[...]

    return pl.pallas_call(
        body,
        grid=(SN // RB,),
        in_specs=[
            pl.BlockSpec((RB, 128), lambda i: (i, 0)),
            pl.BlockSpec((RB, 1), lambda i: (i, 0)),
            pl.BlockSpec((RB, 128), lambda i: (i, 0)),
            pl.BlockSpec((128, 128), lambda i: (0, 0)),
            pl.BlockSpec((1, 128), lambda i: (0, 0)),
            pl.BlockSpec((128, 384), lambda i: (0, 0)),
            pl.BlockSpec((128, 384), lambda i: (0, 0)),
            pl.BlockSpec((1, 384), lambda i: (0, 0)),
            pl.BlockSpec((1, 384), lambda i: (0, 0)),
            pl.BlockSpec((128, 1), lambda i: (0, 0)),
            pl.BlockSpec((128, 1), lambda i: (0, 0)),
            pl.BlockSpec((1, 1), lambda i: (0, 0)),
            pl.BlockSpec((128, 128), lambda i: (0, 0)),
            pl.BlockSpec((1, 128), lambda i: (0, 0)),
        ],
        out_specs=[
            pl.BlockSpec((RB, 128), lambda i: (i, 0)),
            pl.BlockSpec((RB, 128), lambda i: (i, 0)),
            pl.BlockSpec((RB, 1), lambda i: (i, 0)),
            pl.BlockSpec((RB, 1), lambda i: (i, 0)),
        ],
        out_shape=[
            jax.ShapeDtypeStruct((SN, 128), jnp.float32),
            jax.ShapeDtypeStruct((SN, 128), jnp.float32),
            jax.ShapeDtypeStruct((SN, 1), jnp.float32),
            jax.ShapeDtypeStruct((SN, 1), jnp.float32),
        ],
    )(Cw, sw, hvnew, Wet, bet, WihT, WhhT, bih, bhh, u1, u2, bu2,
      Wpnode, bpnode)


def _tck_post_l1(C2, s2, node, WihT, WhhT, bih, bhh,
                 Wpn0, bpn0, Wpn1, bpn1, wb0, bcl0, wb1, bcl1):
    def body(C_r, s_r, nd_r, WihT_r, WhhT_r, bih_r, bhh_r,
             Wpn0_r, bpn0_r, Wpn1_r, bpn1_r, wb0_r, bcl0_r, wb1_r, bcl1_r,
             n2_r, hv0_r, hv1_r, nb0_r, nb1_r):
        c = C_r[...] / (s_r[...] + EPS)
        n2 = jax.nn.relu(_gru_tc(_elu(c), nd_r[...], WihT_r[...],
                                 WhhT_r[...], bih_r[...], bhh_r[...]))
        n2_r[...] = n2
        hv0_r[...] = jnp.dot(n2, Wpn0_r[...],
                             preferred_element_type=jnp.float32) + bpn0_r[...]
        hv1_r[...] = jnp.dot(n2, Wpn1_r[...],
                             preferred_element_type=jnp.float32) + bpn1_r[...]
        nb0_r[...] = jnp.dot(n2, wb0_r[...],
                             preferred_element_type=jnp.float32) + bcl0_r[...]
        nb1_r[...] = jnp.dot(n2, wb1_r[...],
                             preferred_element_type=jnp.float32) + bcl1_r[...]

    return pl.pallas_call(
        body,
        grid=(SN // RB,),
        in_specs=[
            pl.BlockSpec((RB, 128), lambda i: (i, 0)),
            pl.BlockSpec((RB, 1), lambda i: (i, 0)),
            pl.BlockSpec((RB, 128), lambda i: (i, 0)),
            pl.BlockSpec((128, 384), lambda i: (0, 0)),
            pl.BlockSpec((128, 384), lambda i: (0, 0)),
            pl.BlockSpec((1, 384), lambda i: (0, 0)),
            pl.BlockSpec((1, 384), lambda i: (0, 0)),
            pl.BlockSpec((128, 128), lambda i: (0, 0)),
            pl.BlockSpec((1, 128), lambda i: (0, 0)),
            pl.BlockSpec((128, 128), lambda i: (0, 0)),
            pl.BlockSpec((1, 128), lambda i: (0, 0)),
            pl.BlockSpec((128, 1), lambda i: (0, 0)),
            pl.BlockSpec((1, 1), lambda i: (0, 0)),
            pl.BlockSpec((128, 1), lambda i: (0, 0)),
            pl.BlockSpec((1, 1), lambda i: (0, 0)),
        ],
        out_specs=[pl.BlockSpec((RB, 128), lambda i: (i, 0))] * 3
        + [pl.BlockSpec((RB, 1), lambda i: (i, 0))] * 2,
        out_shape=[jax.ShapeDtypeStruct((SN, 128), jnp.float32)] * 3
        + [jax.ShapeDtypeStruct((SN, 1), jnp.float32)] * 2,
    )(C2, s2, node, WihT, WhhT, bih, bhh, Wpn0, bpn0, Wpn1, bpn1,
      wb0, bcl0, wb1, bcl1)


def _tck_tg(G, wa):
    def body(G_r, wa_r, tg_r):
        tg_r[...] = jnp.dot(jax.nn.relu(G_r[...]), wa_r[...],
                            preferred_element_type=jnp.float32)

    return pl.pallas_call(
        body,
        grid=(1,),
        in_specs=[
            pl.BlockSpec((2 * NB, 128), lambda i: (0, 0)),
            pl.BlockSpec((128, 1), lambda i: (0, 0)),
        ],
        out_specs=pl.BlockSpec((2 * NB, 1), lambda i: (0, 0)),
        out_shape=jax.ShapeDtypeStruct((2 * NB, 1), jnp.float32),
    )(G, wa)


def _tck_ro_gru(G, s, h, WihT, WhhT, bih, bhh, wa_next):
    def body(G_r, s_r, h_r, WihT_r, WhhT_r, bih_r, bhh_r, wa_r, g_r, tg_r):
        g_repr = _elu(G_r[...] / (s_r[...] + EPS))
        g = jax.nn.relu(_gru_tc(g_repr, h_r[...], WihT_r[...], WhhT_r[...],
                                bih_r[...], bhh_r[...]))
        g_r[...] = g
        tg_r[...] = jnp.dot(jax.nn.relu(g), wa_r[...],
                            preferred_element_type=jnp.float32)

    return pl.pallas_call(
        body,
        grid=(1,),
        in_specs=[
            pl.BlockSpec((2 * NB, 128), lambda i: (0, 0)),
            pl.BlockSpec((2 * NB, 1), lambda i: (0, 0)),
            pl.BlockSpec((2 * NB, 128), lambda i: (0, 0)),
            pl.BlockSpec((128, 384), lambda i: (0, 0)),
            pl.BlockSpec((128, 384), lambda i: (0, 0)),
            pl.BlockSpec((1, 384), lambda i: (0, 0)),
            pl.BlockSpec((1, 384), lambda i: (0, 0)),
            pl.BlockSpec((128, 1), lambda i: (0, 0)),
        ],
        out_specs=[
            pl.BlockSpec((2 * NB, 128), lambda i: (0, 0)),
            pl.BlockSpec((2 * NB, 1), lambda i: (0, 0)),
        ],
        out_shape=[
            jax.ShapeDtypeStruct((2 * NB, 128), jnp.float32),
            jax.ShapeDtypeStruct((2 * NB, 1), jnp.float32),
        ],
    )(G, s, h, WihT, WhhT, bih, bhh, wa_next)


def _tck_final(G, s, h, WihT, WhhT, bih, bhh, Wpred, bpred,
               Wfc, bfc, bn_a, bn_b, Wout, bout):
    def body(G_r, s_r, h_r, WihT_r, WhhT_r, bih_r, bhh_r, Wp_r, bp_r,
             WA_r, bfc_r, bna_r, bnb_r, Wo_r, bo_r, o_r):
        g_repr = _elu(G_r[...] / (s_r[...] + EPS))
        g = jax.nn.relu(_gru_tc(g_repr, h_r[...], WihT_r[...], WhhT_r[...],
                                bih_r[...], bhh_r[...]))
        pred = jnp.dot(g, Wp_r[...],
                       preferred_element_type=jnp.float32) + bp_r[...]
        s1 = pred[0:B, :]
        s2 = pred[NB:NB + B, :]
        out = jnp.concatenate([s1, s2], axis=1)
        hh = (jnp.dot(out, WA_r[...], preferred_element_type=jnp.float32)
              + bfc_r[...])
        hh = jax.nn.relu(hh * bna_r[...] + bnb_r[...])
        o_r[...] = jnp.dot(hh, Wo_r[...],
                           preferred_element_type=jnp.float32) + bo_r[...]

    return pl.pallas_call(
        body,
        grid=(1,),
        in_specs=[
            pl.BlockSpec((2 * NB, 128), lambda i: (0, 0)),
            pl.BlockSpec((2 * NB, 1), lambda i: (0, 0)),
            pl.BlockSpec((2 * NB, 128), lambda i: (0, 0)),
            pl.BlockSpec((128, 384), lambda i: (0, 0)),
            pl.BlockSpec((128, 384), lambda i: (0, 0)),
            pl.BlockSpec((1, 384), lambda i: (0, 0)),
            pl.BlockSpec((1, 384), lambda i: (0, 0)),
            pl.BlockSpec((128, 256), lambda i: (0, 0)),
            pl.BlockSpec((1, 256), lambda i: (0, 0)),
            pl.BlockSpec((512, 1024), lambda i: (0, 0)),
            pl.BlockSpec((1, 1024), lambda i: (0, 0)),
            pl.BlockSpec((1, 1024), lambda i: (0, 0)),
            pl.BlockSpec((1, 1024), lambda i: (0, 0)),
            pl.BlockSpec((1024, 1), lambda i: (0, 0)),
            pl.BlockSpec((1, 1), lambda i: (0, 0)),
        ],
        out_specs=pl.BlockSpec((B, 1), lambda i: (0, 0)),
        out_shape=jax.ShapeDtypeStruct((B, 1), jnp.float32),
    )(G, s, h, WihT, WhhT, bih, bhh, Wpred, bpred, Wfc, bfc,
      bn_a, bn_b, Wout, bout)


def kernel(x1, e1, edge_index1, gid1, x2, e2, edge_index2, gid2, Wfc, Wout,
           Wpred, bfc, bn_beta, bn_gamma, bn_mean, bn_var, bout, bpred,
           gc_Wet, gc_Whh, gc_Wih, gc_Wpe1, gc_Wpe2, gc_Wpn, gc_bet, gc_bhh,
           gc_bih, gc_bpe1, gc_bpe2, gc_bpn, l1_Whh, l1_Wih, l1_Wpe,
           l1_Wpnode, l1_bhh, l1_bih, l1_bpe, l1_bpnode, ro0_Wcl, ro0_Whh,
           ro0_Wih, ro0_Wpn, ro0_bcl, ro0_bhh, ro0_bih, ro0_bpn, ro1_Wcl,
           ro1_Whh, ro1_Wih, ro1_Wpn, ro1_bcl, ro1_bhh, ro1_bih, ro1_bpn):
    f32 = jnp.float32
    pad_n = NP - N
    pad_e = EP - E
    xs_pad = lambda a: jnp.pad(a, ((0, pad_n), (0, 0)))
    ep2 = lambda a: jnp.pad(a, ((0, pad_e), (0, 0)))
    ep1 = lambda a, v=0: jnp.pad(a, (0, pad_e), constant_values=v)
    x2n = jnp.concatenate([xs_pad(x1), xs_pad(x2)], axis=0)
    eT = jnp.concatenate([ep2(e1), ep2(e2)], axis=0).T
    src_b = jnp.concatenate([ep1(edge_index1[0]),
                             ep1(edge_index2[0]) + NP])
    dst2d = jnp.stack([ep1(edge_index1[1], N),
                       ep1(edge_index2[1], N)])
    gid_pad = lambda g: jnp.pad(g, (0, pad_n), constant_values=B)
    gid2d = jnp.stack([gid_pad(gid1), gid_pad(gid2)])
    z128 = jnp.zeros((NP, 128), f32)
    z1 = jnp.zeros((NP,), f32)

    row = lambda b: b.reshape(1, -1)
    col = lambda w: w.reshape(-1, 1)
    u_gc = col(gc_Wpe2[:128, 0])
    v_gc = col(gc_Wpe2[128:, 0])
    u1_l1 = col(l1_Wpe[:128, 0])
    u2_l1 = col(l1_Wpe[128:, 0])
    wa0, wb0 = col(ro0_Wcl[:128, 0]), col(ro0_Wcl[128:, 0])
    wa1, wb1 = col(ro1_Wcl[:128, 0]), col(ro1_Wcl[128:, 0])
    bn_a = row(bn_gamma / jnp.sqrt(bn_var + 1e-5))
    bn_b = row(bn_beta - bn_mean * bn_gamma / jnp.sqrt(bn_var + 1e-5))

    hv_new, p_gc, xa = _tck_pre(x2n, gc_Wpn, row(gc_bpn), u_gc,
                                gc_Wpe1[:32], row(gc_bpe1))
    xa_src = _sck_gather_rows(xa.reshape(2, NP, 128), src_b, 128)
    he1, q_gc = _tck_edge_mlp(xa_src, eT, gc_Wpe1[32:],
                              v_gc, gc_bpe2.reshape(1, 1))
    p2d_gc = p_gc.reshape(2, NP)
    edge_gc = _sck_edge(gather_rows=False)
    dst3 = dst2d.reshape(2, EP // KC, KC)
    src3 = src_b.reshape(SE // KC, KC)
    Cw, sw, _sp1 = edge_gc(he1, q_gc, p2d_gc, dst3,
                           src3, z128, z1)
    node, hv_l1, p1, p2 = _tck_post_gc(
        Cw.reshape(SN, 128), sw.reshape(SN, 1), hv_new,
        gc_Wet, row(gc_bet), gc_Wih.T, gc_Whh.T, row(gc_bih), row(gc_bhh),
        u1_l1, u2_l1, l1_bpe.reshape(1, 1), l1_Wpnode, row(l1_bpnode))

    edge_l1 = _sck_edge(gather_rows=True)
    dst3b = dst2d.reshape(2, EP // 32, 32)
    src3b = src_b.reshape(SE // 32, 32)
    C2, s2, _sp2 = edge_l1(hv_l1, p2.reshape(2, NP), p1.reshape(2, NP),
                           dst3b, src3b, z128, z1)
    node2, hv0, hv1, nb0, nb1 = _tck_post_l1(
        C2.reshape(SN, 128), s2.reshape(SN, 1), node,
        l1_Wih.T, l1_Whh.T, row(l1_bih), row(l1_bhh),
        ro0_Wpn, row(ro0_bpn), ro1_Wpn, row(ro1_bpn),
        wb0, ro0_bcl.reshape(1, 1), wb1, ro1_bcl.reshape(1, 1))

    zt = jnp.zeros((2, NB), f32)
    zn = jnp.zeros((SN,), f32)
    gf = _sck_nodes(scale=False)(node2, zt, zn, gid2d, z128, z1)
    gfeats = gf.reshape(2 * NB, 128)
    tg0 = _tck_tg(gfeats, wa0)
    ro_k = _sck_nodes(scale=True)
    G0, S0, _sp3 = ro_k(hv0, tg0.reshape(2, NB), nb0.reshape(SN), gid2d, z128, z1)
    gf1, tg1 = _tck_ro_gru(G0.reshape(2 * NB, 128),
                           S0.reshape(2 * NB, 1), gfeats,
                           ro0_Wih.T, ro0_Whh.T, row(ro0_bih), row(ro0_bhh),
                           wa1)
    G1, S1, _sp4 = ro_k(hv1, tg1.reshape(2, NB), nb1.reshape(SN), gid2d, z128, z1)
    o = _tck_final(G1.reshape(2 * NB, 128), S1.reshape(2 * NB, 1),
                   gf1, ro1_Wih.T, ro1_Whh.T, row(ro1_bih), row(ro1_bhh),
                   Wpred, row(bpred), Wfc, row(bfc), bn_a, bn_b,
                   Wout, bout.reshape(1, 1))
    return o.reshape(B)
